# baseline (speedup 1.0000x reference)
import dataclasses
import functools

import jax
import jax.numpy as jnp
from jax import lax
from jax.experimental import pallas as pl
from jax.experimental.pallas import tpu as pltpu
from jax.experimental.pallas import tpu_sc as plsc

F32 = jnp.float32
I32 = jnp.int32

NW = 32
NC = 2


def _mm_body(x_ref, w_ref, b_ref, o1_ref, o2_ref):
    d = x_ref.shape[1]
    acc = jnp.dot(x_ref[...], w_ref[...], preferred_element_type=F32)
    o1_ref[...] = acc[:, :d] + b_ref[...]
    o2_ref[...] = acc[:, d:]


def _matmul(x_pad, wc, b):
    npad, d = x_pad.shape
    blk = 1024
    grid = npad // blk
    return pl.pallas_call(
        _mm_body,
        grid=(grid,),
        in_specs=[
            pl.BlockSpec((blk, d), lambda i: (i, 0)),
            pl.BlockSpec((d, 2 * d), lambda i: (0, 0)),
            pl.BlockSpec((1, d), lambda i: (0, 0)),
        ],
        out_specs=[
            pl.BlockSpec((blk, d), lambda i: (i, 0)),
            pl.BlockSpec((blk, d), lambda i: (i, 0)),
        ],
        out_shape=[
            jax.ShapeDtypeStruct((npad, d), F32),
            jax.ShapeDtypeStruct((npad, d), F32),
        ],
    )(x_pad, wc, b.reshape(1, d))


def _ppf_body(ps_ref, pd_ref, o_ref):
    ps = ps_ref[...]
    pd = pd_ref[...]
    pos_s, n_s = ps[:, 0:3], ps[:, 3:6]
    pos_d, n_d = pd[:, 0:3], pd[:, 3:6]
    pseudo = pos_s - pos_d

    def angle(v1, v2):
        cx = v1[:, 1:2] * v2[:, 2:3] - v1[:, 2:3] * v2[:, 1:2]
        cy = v1[:, 2:3] * v2[:, 0:1] - v1[:, 0:1] * v2[:, 2:3]
        cz = v1[:, 0:1] * v2[:, 1:2] - v1[:, 1:2] * v2[:, 0:1]
        cn = jnp.sqrt(cx * cx + cy * cy + cz * cz)
        dt = (v1[:, 0:1] * v2[:, 0:1] + v1[:, 1:2] * v2[:, 1:2]
              + v1[:, 2:3] * v2[:, 2:3])
        return jnp.arctan2(cn, dt)

    p0 = jnp.sqrt(jnp.sum(pseudo * pseudo, axis=1, keepdims=True))
    p1 = angle(n_d, pseudo)
    p2 = angle(n_s, pseudo)
    p3 = angle(n_d, n_s)
    srcf = ps[:, 7:8]
    dstf = pd[:, 7:8]
    pad = jnp.zeros((ps.shape[0], 10), F32)
    o_ref[...] = jnp.concatenate([p0, p1, p2, p3, srcf, dstf, pad], axis=1)


def _ppf(ps_ext, pd_ext):
    epad = ps_ext.shape[0]
    blk = 2048
    grid = epad // blk
    return pl.pallas_call(
        _ppf_body,
        grid=(grid,),
        in_specs=[
            pl.BlockSpec((blk, 16), lambda i: (i, 0)),
            pl.BlockSpec((blk, 16), lambda i: (i, 0)),
        ],
        out_specs=pl.BlockSpec((blk, 16), lambda i: (i, 0)),
        out_shape=jax.ShapeDtypeStruct((epad, 16), F32),
    )(ps_ext, pd_ext)


def _make_mesh():
    return plsc.VectorSubcoreMesh(core_axis_name="c", subcore_axis_name="s")


def _sc_params():
    cp = pltpu.CompilerParams()
    if "needs_layout_passes" in pltpu.CompilerParams.__dataclass_fields__:
        cp = dataclasses.replace(cp, needs_layout_passes=False)
    if "use_tc_tiling_on_sc" in pltpu.CompilerParams.__dataclass_fields__:
        cp = dataclasses.replace(cp, use_tc_tiling_on_sc=False)
    return cp


def _gather_rows(pn, srcp, dstp, idxp):
    epad = srcp.shape[0]
    nspad = idxp.shape[0]
    ew = epad // NW
    g = 128
    niter = ew // g
    nsw = nspad // NW
    g2 = 80
    niter2 = nsw // g2

    @functools.partial(
        pl.kernel,
        out_type=(
            jax.ShapeDtypeStruct((epad, 16), F32),
            jax.ShapeDtypeStruct((epad, 16), F32),
            jax.ShapeDtypeStruct((nspad, 16), F32),
        ),
        mesh=_make_mesh(),
        compiler_params=_sc_params(),
        scratch_types=[
            pltpu.VMEM((g,), I32),
            pltpu.VMEM((g, 16), F32),
            pltpu.VMEM((g2,), I32),
            pltpu.VMEM((g2, 16), F32),
        ],
    )
    def k(pn_hbm, src_hbm, dst_hbm, idx_hbm, ps_hbm, pd_hbm, pni_hbm,
          ibuf, rbuf, ibuf2, rbuf2):
        wid = lax.axis_index("s") * NC + lax.axis_index("c")
        iota = lax.iota(I32, 16)
        col7 = jnp.full((16,), 7, I32)

        def tagged(idx_src, out_hbm):
            base = wid * ew

            @pl.loop(0, niter)
            def _(it):
                off = base + it * g
                pltpu.sync_copy(idx_src.at[pl.ds(off, g)], ibuf)
                pltpu.sync_copy(pn_hbm.at[ibuf], rbuf)
                for v in range(g // 16):
                    rows = iota + v * 16
                    vals = plsc.bitcast(ibuf[pl.ds(v * 16, 16)], F32)
                    plsc.store_scatter(rbuf, [rows, col7], vals)
                pltpu.sync_copy(rbuf, out_hbm.at[pl.ds(off, g)])

        tagged(src_hbm, ps_hbm)
        tagged(dst_hbm, pd_hbm)

        base2 = wid * nsw

        @pl.loop(0, niter2)
        def _(it):
            off = base2 + it * g2
            pltpu.sync_copy(idx_hbm.at[pl.ds(off, g2)], ibuf2)
            pltpu.sync_copy(pn_hbm.at[ibuf2], rbuf2)
            pltpu.sync_copy(rbuf2, pni_hbm.at[pl.ds(off, g2)])

    return k(pn, srcp, dstp, idxp)


def _segmax(dst, rec, xw2, xw1b, w3flat):
    e = dst.shape[0]
    npad = xw1b.shape[0]
    rng = 800
    nrounds = npad // (rng * NW)
    ch = 1600
    nchunk = e // ch
    g = 64

    @functools.partial(
        pl.kernel,
        out_type=jax.ShapeDtypeStruct((npad, 128), F32),
        mesh=_make_mesh(),
        compiler_params=_sc_params(),
        scratch_types=[
            pltpu.VMEM((rng * 128,), F32),
            pltpu.VMEM((ch,), I32),
            pltpu.VMEM((ch + g,), I32),
            pltpu.VMEM((g, 16), F32),
            pltpu.VMEM((g,), I32),
            pltpu.VMEM((g, 128), F32),
            pltpu.VMEM((512,), F32),
            pltpu.VMEM((32, 128), F32),
        ],
    )
    def k(dst_hbm, rec_hbm, xw2_hbm, xw1b_hbm, w3_hbm, out_hbm,
          acc, scanbuf, eidbuf, recbuf, srcbuf, rowsbuf, w3buf, xbuf):
        wid = lax.axis_index("s") * NC + lax.axis_index("c")
        pltpu.sync_copy(w3_hbm, w3buf)
        w3v = [[w3buf[pl.ds(r * 128 + v * 16, 16)] for v in range(8)]
               for r in range(4)]
        iota = lax.iota(I32, 16)
        col4 = jnp.full((16,), 4, I32)
        neg = jnp.full((16,), -3e38, F32)

        @pl.loop(0, nrounds)
        def _(rnd):
            lo = (rnd * NW + wid) * rng

            @pl.loop(0, rng * 128 // 16)
            def _(i):
                acc[pl.ds(i * 16, 16)] = neg

            def process_group(i, carry):
                pltpu.sync_copy(
                    rec_hbm.at[eidbuf.at[pl.ds(i * g, g)]], recbuf)
                for v in range(g // 16):
                    rows = iota + v * 16
                    sf = plsc.load_gather(recbuf, [rows, col4])
                    srcbuf[pl.ds(v * 16, 16)] = plsc.bitcast(sf, I32)
                pltpu.sync_copy(xw2_hbm.at[srcbuf], rowsbuf)

                @pl.loop(0, g)
                def _(j):
                    prow = recbuf[j, pl.ds(0, 16)]
                    pint = plsc.bitcast(prow, I32)
                    dl = pint[5] - lo
                    p0 = prow[0]
                    p1 = prow[1]
                    p2 = prow[2]
                    p3 = prow[3]
                    ab = dl * 128
                    for v in range(8):
                        u = (rowsbuf[j, pl.ds(v * 16, 16)]
                             + p0 * w3v[0][v] + p1 * w3v[1][v]
                             + p2 * w3v[2][v] + p3 * w3v[3][v])
                        sl = pl.ds(ab + v * 16, 16)
                        acc[sl] = jnp.maximum(acc[sl], u)
                return carry

            @pl.loop(0, nchunk)
            def _(c):
                pltpu.sync_copy(dst_hbm.at[pl.ds(c * ch, ch)], scanbuf)

                def scan_step(v, n):
                    d = scanbuf[pl.ds(v * 16, 16)]
                    m = (d >= lo) & (d < lo + rng)
                    eidv = iota + (c * ch + v * 16)
                    plsc.store_compressed(eidbuf.at[pl.ds(n, 16)], eidv,
                                          mask=m)
                    cnt = plsc.all_reduce_population_count(m)
                    return n + jnp.max(cnt)

                n = lax.fori_loop(0, ch // 16, scan_step, 0)
                padv = jnp.full((16,), eidbuf[pl.ds(0, 16)][0], I32)
                for t in range(g // 16):
                    eidbuf[pl.ds(n + t * 16, 16)] = padv
                ngroups = (n + g - 1) // g
                lax.fori_loop(0, ngroups, process_group, 0)

            @pl.loop(0, rng // 32)
            def _(t):
                row0 = lo + t * 32
                pltpu.sync_copy(xw1b_hbm.at[pl.ds(row0, 32)], xbuf)

                @pl.loop(0, 32)
                def _(rr):
                    for v in range(8):
                        s = acc[pl.ds((t * 32 + rr) * 128 + v * 16, 16)]
                        sl = pl.ds(v * 16, 16)
                        xbuf[rr, sl] = jnp.maximum(xbuf[rr, sl] + s, 0.0)

                pltpu.sync_copy(xbuf, out_hbm.at[pl.ds(row0, 32)])

    return k(dst, rec, xw2, xw1b, w3flat)


def _out_gather(idxp, outfull):
    nspad = idxp.shape[0]
    nsw = nspad // NW
    g = 80
    niter = nsw // g

    @functools.partial(
        pl.kernel,
        out_type=jax.ShapeDtypeStruct((nspad, 128), F32),
        mesh=_make_mesh(),
        compiler_params=_sc_params(),
        scratch_types=[
            pltpu.VMEM((g,), I32),
            pltpu.VMEM((g, 128), F32),
        ],
    )
    def k(idx_hbm, full_hbm, o_hbm, ibuf, rbuf):
        wid = lax.axis_index("s") * NC + lax.axis_index("c")

        @pl.loop(0, niter)
        def _(it):
            off = wid * nsw + it * g
            pltpu.sync_copy(idx_hbm.at[pl.ds(off, g)], ibuf)
            pltpu.sync_copy(full_hbm.at[ibuf], rbuf)
            pltpu.sync_copy(rbuf, o_hbm.at[pl.ds(off, g)])

    return k(idxp, outfull)


def kernel(x, pos, batch, norm, edge_index, idx, W, b):
    n, d = x.shape
    e = edge_index.shape[1]
    ns = idx.shape[0]
    assert d == 128 and n == 50000 and e == 400000 and ns == 12500

    npad = 51200
    epad = 409600
    nspad = 12800

    src = edge_index[0]
    dst = edge_index[1]

    wc = jnp.concatenate([W[:d], W[d:2 * d]], axis=1)
    w3flat = W[2 * d:2 * d + 4].reshape(-1)

    batchf = lax.bitcast_convert_type(batch, F32).reshape(n, 1)
    pn = jnp.concatenate(
        [pos, norm, batchf, jnp.zeros((n, 9), F32)], axis=1)

    x_pad = jnp.concatenate([x, jnp.zeros((npad - n, d), F32)], axis=0)
    srcp = jnp.concatenate([src, jnp.zeros((epad - e,), I32)])
    dstp = jnp.concatenate([dst, jnp.zeros((epad - e,), I32)])
    idxp = jnp.concatenate([idx, jnp.zeros((nspad - ns,), I32)])

    xw1b, xw2 = _matmul(x_pad, wc, b)
    ps_ext, pd_ext, pni = _gather_rows(pn, srcp, dstp, idxp)
    rec = _ppf(ps_ext, pd_ext)
    outfull = _segmax(dst, rec, xw2, xw1b, w3flat)
    outg = _out_gather(idxp, outfull)

    x_out = outg[:ns]
    pos_out = pni[:ns, 0:3]
    batch_out = lax.bitcast_convert_type(pni[:ns, 6], I32)
    return (x_out, pos_out, batch_out, idx)


# speedup vs baseline: 1.6872x; 1.6872x over previous
import dataclasses
import functools

import jax
import jax.numpy as jnp
from jax import lax
from jax.experimental import pallas as pl
from jax.experimental.pallas import tpu as pltpu
from jax.experimental.pallas import tpu_sc as plsc

F32 = jnp.float32
I32 = jnp.int32

NW = 32
NC = 2


def _mm_body(x_ref, w_ref, b_ref, o1_ref, o2_ref):
    d = x_ref.shape[1]
    acc = jnp.dot(x_ref[...], w_ref[...], preferred_element_type=F32)
    o1_ref[...] = acc[:, :d] + b_ref[...]
    o2_ref[...] = acc[:, d:]


def _matmul(x_pad, wc, b):
    npad, d = x_pad.shape
    blk = 1024
    grid = npad // blk
    return pl.pallas_call(
        _mm_body,
        grid=(grid,),
        in_specs=[
            pl.BlockSpec((blk, d), lambda i: (i, 0)),
            pl.BlockSpec((d, 2 * d), lambda i: (0, 0)),
            pl.BlockSpec((1, d), lambda i: (0, 0)),
        ],
        out_specs=[
            pl.BlockSpec((blk, d), lambda i: (i, 0)),
            pl.BlockSpec((blk, d), lambda i: (i, 0)),
        ],
        out_shape=[
            jax.ShapeDtypeStruct((npad, d), F32),
            jax.ShapeDtypeStruct((npad, d), F32),
        ],
    )(x_pad, wc, b.reshape(1, d))


def _ppf_body(ps_ref, pd_ref, o_ref):
    ps = ps_ref[...]
    pd = pd_ref[...]
    pos_s, n_s = ps[:, 0:3], ps[:, 3:6]
    pos_d, n_d = pd[:, 0:3], pd[:, 3:6]
    pseudo = pos_s - pos_d

    def angle(v1, v2):
        cx = v1[:, 1:2] * v2[:, 2:3] - v1[:, 2:3] * v2[:, 1:2]
        cy = v1[:, 2:3] * v2[:, 0:1] - v1[:, 0:1] * v2[:, 2:3]
        cz = v1[:, 0:1] * v2[:, 1:2] - v1[:, 1:2] * v2[:, 0:1]
        cn = jnp.sqrt(cx * cx + cy * cy + cz * cz)
        dt = (v1[:, 0:1] * v2[:, 0:1] + v1[:, 1:2] * v2[:, 1:2]
              + v1[:, 2:3] * v2[:, 2:3])
        return jnp.arctan2(cn, dt)

    p0 = jnp.sqrt(jnp.sum(pseudo * pseudo, axis=1, keepdims=True))
    p1 = angle(n_d, pseudo)
    p2 = angle(n_s, pseudo)
    p3 = angle(n_d, n_s)
    srcf = ps[:, 7:8]
    dstf = pd[:, 7:8]
    pad = jnp.zeros((ps.shape[0], 10), F32)
    o_ref[...] = jnp.concatenate([p0, p1, p2, p3, srcf, dstf, pad], axis=1)


def _ppf(ps_ext, pd_ext):
    epad = ps_ext.shape[0]
    blk = 2048
    grid = epad // blk
    return pl.pallas_call(
        _ppf_body,
        grid=(grid,),
        in_specs=[
            pl.BlockSpec((blk, 16), lambda i: (i, 0)),
            pl.BlockSpec((blk, 16), lambda i: (i, 0)),
        ],
        out_specs=pl.BlockSpec((blk, 16), lambda i: (i, 0)),
        out_shape=jax.ShapeDtypeStruct((epad, 16), F32),
    )(ps_ext, pd_ext)


def _make_mesh():
    return plsc.VectorSubcoreMesh(core_axis_name="c", subcore_axis_name="s")


def _sc_params():
    cp = pltpu.CompilerParams()
    if "needs_layout_passes" in pltpu.CompilerParams.__dataclass_fields__:
        cp = dataclasses.replace(cp, needs_layout_passes=False)
    if "use_tc_tiling_on_sc" in pltpu.CompilerParams.__dataclass_fields__:
        cp = dataclasses.replace(cp, use_tc_tiling_on_sc=False)
    return cp


def _gather_rows(pn, srcp, dstp, idxp):
    epad = srcp.shape[0]
    nspad = idxp.shape[0]
    ew = epad // NW
    g = 512
    niter = ew // g
    nsw = nspad // NW
    g2 = 400
    niter2 = nsw // g2

    @functools.partial(
        pl.kernel,
        out_type=(
            jax.ShapeDtypeStruct((epad, 16), F32),
            jax.ShapeDtypeStruct((epad, 16), F32),
            jax.ShapeDtypeStruct((nspad, 16), F32),
        ),
        mesh=_make_mesh(),
        compiler_params=_sc_params(),
        scratch_types=[
            pltpu.VMEM((g,), I32),
            pltpu.VMEM((g, 16), F32),
            pltpu.VMEM((g2,), I32),
            pltpu.VMEM((g2, 16), F32),
        ],
    )
    def k(pn_hbm, src_hbm, dst_hbm, idx_hbm, ps_hbm, pd_hbm, pni_hbm,
          ibuf, rbuf, ibuf2, rbuf2):
        wid = lax.axis_index("s") * NC + lax.axis_index("c")
        iota = lax.iota(I32, 16)
        col7 = jnp.full((16,), 7, I32)

        def tagged(idx_src, out_hbm):
            base = wid * ew

            @pl.loop(0, niter)
            def _(it):
                off = base + it * g
                pltpu.sync_copy(idx_src.at[pl.ds(off, g)], ibuf)
                pltpu.sync_copy(pn_hbm.at[ibuf], rbuf)
                for v in range(g // 16):
                    rows = iota + v * 16
                    vals = plsc.bitcast(ibuf[pl.ds(v * 16, 16)], F32)
                    plsc.store_scatter(rbuf, [rows, col7], vals)
                pltpu.sync_copy(rbuf, out_hbm.at[pl.ds(off, g)])

        tagged(src_hbm, ps_hbm)
        tagged(dst_hbm, pd_hbm)

        base2 = wid * nsw

        @pl.loop(0, niter2)
        def _(it):
            off = base2 + it * g2
            pltpu.sync_copy(idx_hbm.at[pl.ds(off, g2)], ibuf2)
            pltpu.sync_copy(pn_hbm.at[ibuf2], rbuf2)
            pltpu.sync_copy(rbuf2, pni_hbm.at[pl.ds(off, g2)])

    return k(pn, srcp, dstp, idxp)


def _segmax(dst, rec, xw2, xw1b, w3flat):
    e = dst.shape[0]
    npad = xw1b.shape[0]
    rng = 784
    nrounds = npad // (rng * NW)
    ch = 1600
    nchunk = e // ch
    g = 128

    @functools.partial(
        pl.kernel,
        out_type=jax.ShapeDtypeStruct((npad, 128), F32),
        mesh=_make_mesh(),
        compiler_params=_sc_params(),
        scratch_types=[
            pltpu.VMEM((rng * 128,), F32),
            pltpu.VMEM((ch,), I32),
            pltpu.VMEM((ch,), I32),
            pltpu.VMEM((1856,), I32),
            pltpu.VMEM((g, 16), F32),
            pltpu.VMEM((g,), I32),
            pltpu.VMEM((g, 128), F32),
            pltpu.VMEM((512,), F32),
            pltpu.VMEM((16, 128), F32),
            pltpu.SemaphoreType.DMA,
            pltpu.SemaphoreType.DMA,
        ],
    )
    def k(dst_hbm, rec_hbm, xw2_hbm, xw1b_hbm, w3_hbm, out_hbm,
          acc, s0, s1, eidbuf, recbuf, srcbuf, rowsbuf, w3buf, xbuf,
          sem0, sem1):
        wid = lax.axis_index("s") * NC + lax.axis_index("c")
        pltpu.sync_copy(w3_hbm, w3buf)
        w3v = [[w3buf[pl.ds(r * 128 + v * 16, 16)] for v in range(8)]
               for r in range(4)]
        iota = lax.iota(I32, 16)
        col4 = jnp.full((16,), 4, I32)
        neg = jnp.full((16,), -3e38, F32)

        @pl.loop(0, nrounds)
        def _(rnd):
            lo = (rnd * NW + wid) * rng

            @pl.loop(0, rng * 128 // 16)
            def _(i):
                acc[pl.ds(i * 16, 16)] = neg

            def process_group(i, carry):
                pltpu.sync_copy(
                    rec_hbm.at[eidbuf.at[pl.ds(i * g, g)]], recbuf)
                for v in range(g // 16):
                    sf = plsc.load_gather(recbuf, [iota + v * 16, col4])
                    srcbuf[pl.ds(v * 16, 16)] = plsc.bitcast(sf, I32)
                pltpu.sync_copy(xw2_hbm.at[srcbuf], rowsbuf)

                @pl.loop(0, g)
                def _(j):
                    prow = recbuf[j, pl.ds(0, 16)]
                    pint = plsc.bitcast(prow, I32)
                    dl = pint[5] - lo
                    p0 = prow[0]
                    p1 = prow[1]
                    p2 = prow[2]
                    p3 = prow[3]
                    ab = dl * 128
                    for v in range(8):
                        u = (rowsbuf[j, pl.ds(v * 16, 16)]
                             + p0 * w3v[0][v] + p1 * w3v[1][v]
                             + p2 * w3v[2][v] + p3 * w3v[3][v])
                        sl = pl.ds(ab + v * 16, 16)
                        acc[sl] = jnp.maximum(acc[sl], u)
                return carry

            pltpu.make_async_copy(
                dst_hbm.at[pl.ds(0, ch)], s0, sem0).start()

            def chunk(c, n, sb, sem, sbn, semn):
                @pl.when(c + 1 < nchunk)
                def _():
                    pltpu.make_async_copy(
                        dst_hbm.at[pl.ds((c + 1) * ch, ch)], sbn,
                        semn).start()
                pltpu.make_async_copy(
                    dst_hbm.at[pl.ds(c * ch, ch)], sb, sem).wait()

                def scan_step(v, nn):
                    d = sb[pl.ds(v * 16, 16)]
                    m = (d >= lo) & (d < lo + rng)
                    eidv = iota + (c * ch + v * 16)
                    plsc.store_compressed(eidbuf.at[pl.ds(nn, 16)], eidv,
                                          mask=m)
                    cnt = plsc.all_reduce_population_count(m)
                    return nn + jnp.max(cnt)

                n2 = lax.fori_loop(0, ch // 16, scan_step, n)
                nfull = n2 // g
                lax.fori_loop(0, nfull, process_group, 0)

                @pl.when(nfull > 0)
                def _():
                    for t in range(g // 16):
                        eidbuf[pl.ds(t * 16, 16)] = (
                            eidbuf[pl.ds(nfull * g + t * 16, 16)])
                return n2 - nfull * g

            def pair(p, n):
                n = chunk(2 * p, n, s0, sem0, s1, sem1)
                n = chunk(2 * p + 1, n, s1, sem1, s0, sem0)
                return n

            n = lax.fori_loop(0, nchunk // 2, pair, 0)

            padv = jnp.full((16,), eidbuf[pl.ds(0, 16)][0], I32)
            for t in range(g // 16):
                eidbuf[pl.ds(n + t * 16, 16)] = padv
            lax.fori_loop(0, (n + g - 1) // g, process_group, 0)

            @pl.loop(0, rng // 16)
            def _(t):
                row0 = lo + t * 16
                pltpu.sync_copy(xw1b_hbm.at[pl.ds(row0, 16)], xbuf)

                @pl.loop(0, 16)
                def _(rr):
                    for v in range(8):
                        s = acc[pl.ds((t * 16 + rr) * 128 + v * 16, 16)]
                        sl = pl.ds(v * 16, 16)
                        xbuf[rr, sl] = jnp.maximum(xbuf[rr, sl] + s, 0.0)

                pltpu.sync_copy(xbuf, out_hbm.at[pl.ds(row0, 16)])

    return k(dst, rec, xw2, xw1b, w3flat)


def _out_gather(idxp, outfull):
    nspad = idxp.shape[0]
    nsw = nspad // NW
    g = 400
    niter = nsw // g

    @functools.partial(
        pl.kernel,
        out_type=jax.ShapeDtypeStruct((nspad, 128), F32),
        mesh=_make_mesh(),
        compiler_params=_sc_params(),
        scratch_types=[
            pltpu.VMEM((g,), I32),
            pltpu.VMEM((g, 128), F32),
        ],
    )
    def k(idx_hbm, full_hbm, o_hbm, ibuf, rbuf):
        wid = lax.axis_index("s") * NC + lax.axis_index("c")

        @pl.loop(0, niter)
        def _(it):
            off = wid * nsw + it * g
            pltpu.sync_copy(idx_hbm.at[pl.ds(off, g)], ibuf)
            pltpu.sync_copy(full_hbm.at[ibuf], rbuf)
            pltpu.sync_copy(rbuf, o_hbm.at[pl.ds(off, g)])

    return k(idxp, outfull)


def kernel(x, pos, batch, norm, edge_index, idx, W, b):
    n, d = x.shape
    e = edge_index.shape[1]
    ns = idx.shape[0]
    assert d == 128 and n == 50000 and e == 400000 and ns == 12500

    npad = 50176
    epad = 409600
    nspad = 12800

    src = edge_index[0]
    dst = edge_index[1]

    wc = jnp.concatenate([W[:d], W[d:2 * d]], axis=1)
    w3flat = W[2 * d:2 * d + 4].reshape(-1)

    batchf = lax.bitcast_convert_type(batch, F32).reshape(n, 1)
    pn = jnp.concatenate(
        [pos, norm, batchf, jnp.zeros((n, 9), F32)], axis=1)

    x_pad = jnp.concatenate([x, jnp.zeros((npad - n, d), F32)], axis=0)
    srcp = jnp.concatenate([src, jnp.zeros((epad - e,), I32)])
    dstp = jnp.concatenate([dst, jnp.zeros((epad - e,), I32)])
    idxp = jnp.concatenate([idx, jnp.zeros((nspad - ns,), I32)])

    xw1b, xw2 = _matmul(x_pad, wc, b)
    ps_ext, pd_ext, pni = _gather_rows(pn, srcp, dstp, idxp)
    rec = _ppf(ps_ext, pd_ext)
    outfull = _segmax(dst, rec, xw2, xw1b, w3flat)
    outg = _out_gather(idxp, outfull)

    x_out = outg[:ns]
    pos_out = pni[:ns, 0:3]
    batch_out = lax.bitcast_convert_type(pni[:ns, 6], I32)
    return (x_out, pos_out, batch_out, idx)


# speedup vs baseline: 2.8049x; 1.6625x over previous
import dataclasses
import functools

import jax
import jax.numpy as jnp
from jax import lax
from jax.experimental import pallas as pl
from jax.experimental.pallas import tpu as pltpu
from jax.experimental.pallas import tpu_sc as plsc

F32 = jnp.float32
I32 = jnp.int32

NW = 32
NC = 2


def _mm_body(x_ref, w_ref, b_ref, o1_ref, o2_ref):
    d = x_ref.shape[1]
    acc = jnp.dot(x_ref[...], w_ref[...], preferred_element_type=F32)
    o1_ref[...] = acc[:, :d] + b_ref[...]
    o2_ref[...] = acc[:, d:]


def _matmul(x_pad, wc, b):
    npad, d = x_pad.shape
    blk = 1024
    grid = npad // blk
    return pl.pallas_call(
        _mm_body,
        grid=(grid,),
        in_specs=[
            pl.BlockSpec((blk, d), lambda i: (i, 0)),
            pl.BlockSpec((d, 2 * d), lambda i: (0, 0)),
            pl.BlockSpec((1, d), lambda i: (0, 0)),
        ],
        out_specs=[
            pl.BlockSpec((blk, d), lambda i: (i, 0)),
            pl.BlockSpec((blk, d), lambda i: (i, 0)),
        ],
        out_shape=[
            jax.ShapeDtypeStruct((npad, d), F32),
            jax.ShapeDtypeStruct((npad, d), F32),
        ],
    )(x_pad, wc, b.reshape(1, d))


def _ppf_body(ps_ref, pd_ref, o_ref):
    ps = ps_ref[...]
    pd = pd_ref[...]

    def cross_dot(v1, v2):
        cx = v1[1:2] * v2[2:3] - v1[2:3] * v2[1:2]
        cy = v1[2:3] * v2[0:1] - v1[0:1] * v2[2:3]
        cz = v1[0:1] * v2[1:2] - v1[1:2] * v2[0:1]
        cn = jnp.sqrt(cx * cx + cy * cy + cz * cz)
        dt = v1[0:1] * v2[0:1] + v1[1:2] * v2[1:2] + v1[2:3] * v2[2:3]
        return jnp.arctan2(cn, dt)

    pos_s, n_s = ps[0:3], ps[3:6]
    pos_d, n_d = pd[0:3], pd[3:6]
    pseudo = pos_s - pos_d
    p0 = jnp.sqrt(jnp.sum(pseudo * pseudo, axis=0, keepdims=True))
    p1 = cross_dot(n_d, pseudo)
    p2 = cross_dot(n_s, pseudo)
    p3 = cross_dot(n_d, n_s)
    zero = jnp.zeros((10, ps.shape[1]), F32)
    o_ref[...] = jnp.concatenate(
        [p0, p1, p2, p3, ps[7:8], pd[7:8], zero], axis=0)


def _ppf(ps_t, pd_t):
    epad = ps_t.shape[1]
    blk = 1024
    grid = epad // blk
    return pl.pallas_call(
        _ppf_body,
        grid=(grid,),
        in_specs=[
            pl.BlockSpec((16, blk), lambda i: (0, i)),
            pl.BlockSpec((16, blk), lambda i: (0, i)),
        ],
        out_specs=pl.BlockSpec((16, blk), lambda i: (0, i)),
        out_shape=jax.ShapeDtypeStruct((16, epad), F32),
    )(ps_t, pd_t)


def _make_mesh():
    return plsc.VectorSubcoreMesh(core_axis_name="c", subcore_axis_name="s")


def _sc_params():
    cp = pltpu.CompilerParams()
    if "needs_layout_passes" in pltpu.CompilerParams.__dataclass_fields__:
        cp = dataclasses.replace(cp, needs_layout_passes=False)
    if "use_tc_tiling_on_sc" in pltpu.CompilerParams.__dataclass_fields__:
        cp = dataclasses.replace(cp, use_tc_tiling_on_sc=False)
    return cp


def _gather_rows(pn, srcp, dstp, idxp):
    epad = srcp.shape[0]
    nspad = idxp.shape[0]
    ew = epad // NW
    g = 512
    niter = ew // g
    nsw = nspad // NW
    g2 = 400
    niter2 = nsw // g2

    @functools.partial(
        pl.kernel,
        out_type=(
            jax.ShapeDtypeStruct((epad, 16), F32),
            jax.ShapeDtypeStruct((epad, 16), F32),
            jax.ShapeDtypeStruct((nspad, 16), F32),
        ),
        mesh=_make_mesh(),
        compiler_params=_sc_params(),
        scratch_types=[
            pltpu.VMEM((g,), I32),
            pltpu.VMEM((g, 16), F32),
            pltpu.VMEM((g2,), I32),
            pltpu.VMEM((g2, 16), F32),
        ],
    )
    def k(pn_hbm, src_hbm, dst_hbm, idx_hbm, ps_hbm, pd_hbm, pni_hbm,
          ibuf, rbuf, ibuf2, rbuf2):
        wid = lax.axis_index("s") * NC + lax.axis_index("c")
        iota = lax.iota(I32, 16)
        col7 = jnp.full((16,), 7, I32)

        def tagged(idx_src, out_hbm):
            base = wid * ew

            @pl.loop(0, niter)
            def _(it):
                off = base + it * g
                pltpu.sync_copy(idx_src.at[pl.ds(off, g)], ibuf)
                pltpu.sync_copy(pn_hbm.at[ibuf], rbuf)
                for v in range(g // 16):
                    rows = iota + v * 16
                    vals = plsc.bitcast(ibuf[pl.ds(v * 16, 16)], F32)
                    plsc.store_scatter(rbuf, [rows, col7], vals)
                pltpu.sync_copy(rbuf, out_hbm.at[pl.ds(off, g)])

        tagged(src_hbm, ps_hbm)
        tagged(dst_hbm, pd_hbm)

        base2 = wid * nsw

        @pl.loop(0, niter2)
        def _(it):
            off = base2 + it * g2
            pltpu.sync_copy(idx_hbm.at[pl.ds(off, g2)], ibuf2)
            pltpu.sync_copy(pn_hbm.at[ibuf2], rbuf2)
            pltpu.sync_copy(rbuf2, pni_hbm.at[pl.ds(off, g2)])

    return k(pn, srcp, dstp, idxp)


def _segmax(dst, rec, xw2, xw1b, w3flat):
    e = dst.shape[0]
    npad = xw1b.shape[0]
    rng = 784
    nrounds = npad // (rng * NW)
    ch = 1600
    nchunk = e // ch
    g = 128

    @functools.partial(
        pl.kernel,
        out_type=jax.ShapeDtypeStruct((npad, 128), F32),
        mesh=_make_mesh(),
        compiler_params=_sc_params(),
        scratch_types=[
            pltpu.VMEM((rng * 128,), F32),
            pltpu.VMEM((ch,), I32),
            pltpu.VMEM((ch,), I32),
            pltpu.VMEM((1856,), I32),
            pltpu.VMEM((g, 16), F32),
            pltpu.VMEM((g,), I32),
            pltpu.VMEM((g, 128), F32),
            pltpu.VMEM((512,), F32),
            pltpu.VMEM((16, 128), F32),
            pltpu.SemaphoreType.DMA,
            pltpu.SemaphoreType.DMA,
        ],
    )
    def k(dst_hbm, rec_hbm, xw2_hbm, xw1b_hbm, w3_hbm, out_hbm,
          acc, s0, s1, eidbuf, recbuf, srcbuf, rowsbuf, w3buf, xbuf,
          sem0, sem1):
        wid = lax.axis_index("s") * NC + lax.axis_index("c")
        pltpu.sync_copy(w3_hbm, w3buf)
        w3v = [[w3buf[pl.ds(r * 128 + v * 16, 16)] for v in range(8)]
               for r in range(4)]
        iota = lax.iota(I32, 16)
        col4 = jnp.full((16,), 4, I32)
        neg = jnp.full((16,), -3e38, F32)

        @pl.loop(0, nrounds)
        def _(rnd):
            lo = (rnd * NW + wid) * rng

            @pl.loop(0, rng * 128 // 16)
            def _(i):
                acc[pl.ds(i * 16, 16)] = neg

            def process_group(i, carry):
                pltpu.sync_copy(
                    rec_hbm.at[eidbuf.at[pl.ds(i * g, g)]], recbuf)
                for v in range(g // 16):
                    sf = plsc.load_gather(recbuf, [iota + v * 16, col4])
                    srcbuf[pl.ds(v * 16, 16)] = plsc.bitcast(sf, I32)
                pltpu.sync_copy(xw2_hbm.at[srcbuf], rowsbuf)

                @pl.loop(0, g)
                def _(j):
                    prow = recbuf[j, pl.ds(0, 16)]
                    pint = plsc.bitcast(prow, I32)
                    dl = pint[5] - lo
                    p0 = prow[0]
                    p1 = prow[1]
                    p2 = prow[2]
                    p3 = prow[3]
                    ab = dl * 128
                    for v in range(8):
                        u = (rowsbuf[j, pl.ds(v * 16, 16)]
                             + p0 * w3v[0][v] + p1 * w3v[1][v]
                             + p2 * w3v[2][v] + p3 * w3v[3][v])
                        sl = pl.ds(ab + v * 16, 16)
                        acc[sl] = jnp.maximum(acc[sl], u)
                return carry

            pltpu.make_async_copy(
                dst_hbm.at[pl.ds(0, ch)], s0, sem0).start()

            def chunk(c, n, sb, sem, sbn, semn):
                @pl.when(c + 1 < nchunk)
                def _():
                    pltpu.make_async_copy(
                        dst_hbm.at[pl.ds((c + 1) * ch, ch)], sbn,
                        semn).start()
                pltpu.make_async_copy(
                    dst_hbm.at[pl.ds(c * ch, ch)], sb, sem).wait()

                def scan_step(v, nn):
                    d = sb[pl.ds(v * 16, 16)]
                    m = (d >= lo) & (d < lo + rng)
                    eidv = iota + (c * ch + v * 16)
                    plsc.store_compressed(eidbuf.at[pl.ds(nn, 16)], eidv,
                                          mask=m)
                    cnt = plsc.all_reduce_population_count(m)
                    return nn + jnp.max(cnt)

                n2 = lax.fori_loop(0, ch // 16, scan_step, n)
                nfull = n2 // g
                lax.fori_loop(0, nfull, process_group, 0)

                @pl.when(nfull > 0)
                def _():
                    for t in range(g // 16):
                        eidbuf[pl.ds(t * 16, 16)] = (
                            eidbuf[pl.ds(nfull * g + t * 16, 16)])
                return n2 - nfull * g

            def pair(p, n):
                n = chunk(2 * p, n, s0, sem0, s1, sem1)
                n = chunk(2 * p + 1, n, s1, sem1, s0, sem0)
                return n

            n = lax.fori_loop(0, nchunk // 2, pair, 0)

            padv = jnp.full((16,), eidbuf[pl.ds(0, 16)][0], I32)
            for t in range(g // 16):
                eidbuf[pl.ds(n + t * 16, 16)] = padv
            lax.fori_loop(0, (n + g - 1) // g, process_group, 0)

            @pl.loop(0, rng // 16)
            def _(t):
                row0 = lo + t * 16
                pltpu.sync_copy(xw1b_hbm.at[pl.ds(row0, 16)], xbuf)

                @pl.loop(0, 16)
                def _(rr):
                    for v in range(8):
                        s = acc[pl.ds((t * 16 + rr) * 128 + v * 16, 16)]
                        sl = pl.ds(v * 16, 16)
                        xbuf[rr, sl] = jnp.maximum(xbuf[rr, sl] + s, 0.0)

                pltpu.sync_copy(xbuf, out_hbm.at[pl.ds(row0, 16)])

    return k(dst, rec, xw2, xw1b, w3flat)


def _out_gather(idxp, outfull):
    nspad = idxp.shape[0]
    nsw = nspad // NW
    g = 400
    niter = nsw // g

    @functools.partial(
        pl.kernel,
        out_type=jax.ShapeDtypeStruct((nspad, 128), F32),
        mesh=_make_mesh(),
        compiler_params=_sc_params(),
        scratch_types=[
            pltpu.VMEM((g,), I32),
            pltpu.VMEM((g, 128), F32),
        ],
    )
    def k(idx_hbm, full_hbm, o_hbm, ibuf, rbuf):
        wid = lax.axis_index("s") * NC + lax.axis_index("c")

        @pl.loop(0, niter)
        def _(it):
            off = wid * nsw + it * g
            pltpu.sync_copy(idx_hbm.at[pl.ds(off, g)], ibuf)
            pltpu.sync_copy(full_hbm.at[ibuf], rbuf)
            pltpu.sync_copy(rbuf, o_hbm.at[pl.ds(off, g)])

    return k(idxp, outfull)


def kernel(x, pos, batch, norm, edge_index, idx, W, b):
    n, d = x.shape
    e = edge_index.shape[1]
    ns = idx.shape[0]
    assert d == 128 and n == 50000 and e == 400000 and ns == 12500

    npad = 50176
    epad = 409600
    nspad = 12800

    src = edge_index[0]
    dst = edge_index[1]

    wc = jnp.concatenate([W[:d], W[d:2 * d]], axis=1)
    w3flat = W[2 * d:2 * d + 4].reshape(-1)

    batchf = lax.bitcast_convert_type(batch, F32).reshape(n, 1)
    pn = jnp.concatenate(
        [pos, norm, batchf, jnp.zeros((n, 9), F32)], axis=1)

    x_pad = jnp.concatenate([x, jnp.zeros((npad - n, d), F32)], axis=0)
    srcp = jnp.concatenate([src, jnp.zeros((epad - e,), I32)])
    dstp = jnp.concatenate([dst, jnp.zeros((epad - e,), I32)])
    idxp = jnp.concatenate([idx, jnp.zeros((nspad - ns,), I32)])

    xw1b, xw2 = _matmul(x_pad, wc, b)
    ps_ext, pd_ext, pni = _gather_rows(pn, srcp, dstp, idxp)
    rec = _ppf(ps_ext.T, pd_ext.T).T
    outfull = _segmax(dst, rec, xw2, xw1b, w3flat)
    outg = _out_gather(idxp, outfull)

    x_out = outg[:ns]
    pos_out = pni[:ns, 0:3]
    batch_out = lax.bitcast_convert_type(pni[:ns, 6], I32)
    return (x_out, pos_out, batch_out, idx)


# speedup vs baseline: 2.8055x; 1.0002x over previous
import dataclasses
import functools

import jax
import jax.numpy as jnp
from jax import lax
from jax.experimental import pallas as pl
from jax.experimental.pallas import tpu as pltpu
from jax.experimental.pallas import tpu_sc as plsc

F32 = jnp.float32
I32 = jnp.int32

NW = 32
NC = 2


def _mm_body(x_ref, w_ref, b_ref, o1_ref, o2_ref):
    d = x_ref.shape[1]
    acc = jnp.dot(x_ref[...], w_ref[...], preferred_element_type=F32)
    o1_ref[...] = acc[:, :d] + b_ref[...]
    o2_ref[...] = acc[:, d:]


def _matmul(x_pad, wc, b):
    npad, d = x_pad.shape
    blk = 1024
    grid = npad // blk
    return pl.pallas_call(
        _mm_body,
        grid=(grid,),
        in_specs=[
            pl.BlockSpec((blk, d), lambda i: (i, 0)),
            pl.BlockSpec((d, 2 * d), lambda i: (0, 0)),
            pl.BlockSpec((1, d), lambda i: (0, 0)),
        ],
        out_specs=[
            pl.BlockSpec((blk, d), lambda i: (i, 0)),
            pl.BlockSpec((blk, d), lambda i: (i, 0)),
        ],
        out_shape=[
            jax.ShapeDtypeStruct((npad, d), F32),
            jax.ShapeDtypeStruct((npad, d), F32),
        ],
    )(x_pad, wc, b.reshape(1, d))


def _ppf_body(ps_ref, pd_ref, o_ref):
    ps = ps_ref[...]
    pd = pd_ref[...]

    def cross_dot(v1, v2):
        cx = v1[1:2] * v2[2:3] - v1[2:3] * v2[1:2]
        cy = v1[2:3] * v2[0:1] - v1[0:1] * v2[2:3]
        cz = v1[0:1] * v2[1:2] - v1[1:2] * v2[0:1]
        cn = jnp.sqrt(cx * cx + cy * cy + cz * cz)
        dt = v1[0:1] * v2[0:1] + v1[1:2] * v2[1:2] + v1[2:3] * v2[2:3]
        return jnp.arctan2(cn, dt)

    pos_s, n_s = ps[0:3], ps[3:6]
    pos_d, n_d = pd[0:3], pd[3:6]
    pseudo = pos_s - pos_d
    p0 = jnp.sqrt(jnp.sum(pseudo * pseudo, axis=0, keepdims=True))
    p1 = cross_dot(n_d, pseudo)
    p2 = cross_dot(n_s, pseudo)
    p3 = cross_dot(n_d, n_s)
    zero = jnp.zeros((10, ps.shape[1]), F32)
    o_ref[...] = jnp.concatenate(
        [p0, p1, p2, p3, ps[7:8], pd[7:8], zero], axis=0)


def _ppf(ps_t, pd_t):
    epad = ps_t.shape[1]
    blk = 1024
    grid = epad // blk
    return pl.pallas_call(
        _ppf_body,
        grid=(grid,),
        in_specs=[
            pl.BlockSpec((16, blk), lambda i: (0, i)),
            pl.BlockSpec((16, blk), lambda i: (0, i)),
        ],
        out_specs=pl.BlockSpec((16, blk), lambda i: (0, i)),
        out_shape=jax.ShapeDtypeStruct((16, epad), F32),
    )(ps_t, pd_t)


def _make_mesh():
    return plsc.VectorSubcoreMesh(core_axis_name="c", subcore_axis_name="s")


def _sc_params():
    cp = pltpu.CompilerParams()
    if "needs_layout_passes" in pltpu.CompilerParams.__dataclass_fields__:
        cp = dataclasses.replace(cp, needs_layout_passes=False)
    if "use_tc_tiling_on_sc" in pltpu.CompilerParams.__dataclass_fields__:
        cp = dataclasses.replace(cp, use_tc_tiling_on_sc=False)
    return cp


def _gather_rows(pn, srcp, dstp, idxp):
    epad = srcp.shape[0]
    nspad = idxp.shape[0]
    ew = epad // NW
    g = 512
    niter = ew // g
    nsw = nspad // NW
    g2 = 400
    niter2 = nsw // g2

    @functools.partial(
        pl.kernel,
        out_type=(
            jax.ShapeDtypeStruct((epad, 16), F32),
            jax.ShapeDtypeStruct((epad, 16), F32),
            jax.ShapeDtypeStruct((nspad, 16), F32),
        ),
        mesh=_make_mesh(),
        compiler_params=_sc_params(),
        scratch_types=[
            pltpu.VMEM((g,), I32),
            pltpu.VMEM((g, 16), F32),
            pltpu.VMEM((g2,), I32),
            pltpu.VMEM((g2, 16), F32),
        ],
    )
    def k(pn_hbm, src_hbm, dst_hbm, idx_hbm, ps_hbm, pd_hbm, pni_hbm,
          ibuf, rbuf, ibuf2, rbuf2):
        wid = lax.axis_index("s") * NC + lax.axis_index("c")
        iota = lax.iota(I32, 16)
        col7 = jnp.full((16,), 7, I32)

        def tagged(idx_src, out_hbm):
            base = wid * ew

            @pl.loop(0, niter)
            def _(it):
                off = base + it * g
                pltpu.sync_copy(idx_src.at[pl.ds(off, g)], ibuf)
                pltpu.sync_copy(pn_hbm.at[ibuf], rbuf)
                for v in range(g // 16):
                    rows = iota + v * 16
                    vals = plsc.bitcast(ibuf[pl.ds(v * 16, 16)], F32)
                    plsc.store_scatter(rbuf, [rows, col7], vals)
                pltpu.sync_copy(rbuf, out_hbm.at[pl.ds(off, g)])

        tagged(src_hbm, ps_hbm)
        tagged(dst_hbm, pd_hbm)

        base2 = wid * nsw

        @pl.loop(0, niter2)
        def _(it):
            off = base2 + it * g2
            pltpu.sync_copy(idx_hbm.at[pl.ds(off, g2)], ibuf2)
            pltpu.sync_copy(pn_hbm.at[ibuf2], rbuf2)
            pltpu.sync_copy(rbuf2, pni_hbm.at[pl.ds(off, g2)])

    return k(pn, srcp, dstp, idxp)


def _segmax(dst, rec, xw2, xw1b, w3flat):
    e = dst.shape[0]
    npad = xw1b.shape[0]
    rng = 784
    nrounds = npad // (rng * NW)
    ch = 1600
    nchunk = e // ch
    g = 128

    @functools.partial(
        pl.kernel,
        out_type=jax.ShapeDtypeStruct((npad, 128), F32),
        mesh=_make_mesh(),
        compiler_params=_sc_params(),
        scratch_types=[
            pltpu.VMEM((rng * 128,), F32),
            pltpu.VMEM((ch,), I32),
            pltpu.VMEM((ch,), I32),
            pltpu.VMEM((1856,), I32),
            pltpu.VMEM((g, 16), F32),
            pltpu.VMEM((g,), I32),
            pltpu.VMEM((g, 128), F32),
            pltpu.VMEM((512,), F32),
            pltpu.VMEM((32, 128), F32),
            pltpu.SemaphoreType.DMA,
            pltpu.SemaphoreType.DMA,
            pltpu.SemaphoreType.DMA,
            pltpu.SemaphoreType.DMA,
            pltpu.SemaphoreType.DMA,
            pltpu.SemaphoreType.DMA,
        ],
    )
    def k(dst_hbm, rec_hbm, xw2_hbm, xw1b_hbm, w3_hbm, out_hbm,
          acc, s0, s1, eidbuf, recbuf, srcbuf, rowsbuf, w3buf, xbuf,
          sem0, sem1, semr0, semr1, semw0, semw1):
        wid = lax.axis_index("s") * NC + lax.axis_index("c")
        pltpu.sync_copy(w3_hbm, w3buf)
        w3v = [[w3buf[pl.ds(r * 128 + v * 16, 16)] for v in range(8)]
               for r in range(4)]
        iota = lax.iota(I32, 16)
        col4 = jnp.full((16,), 4, I32)
        neg = jnp.full((16,), -3e38, F32)

        @pl.loop(0, nrounds)
        def _(rnd):
            lo = (rnd * NW + wid) * rng

            @pl.loop(0, rng * 128 // 16)
            def _(i):
                acc[pl.ds(i * 16, 16)] = neg

            def process_group(i, carry):
                pltpu.sync_copy(
                    rec_hbm.at[eidbuf.at[pl.ds(i * g, g)]], recbuf)
                for v in range(g // 16):
                    sf = plsc.load_gather(recbuf, [iota + v * 16, col4])
                    srcbuf[pl.ds(v * 16, 16)] = plsc.bitcast(sf, I32)
                pltpu.sync_copy(xw2_hbm.at[srcbuf], rowsbuf)

                @pl.loop(0, g)
                def _(j):
                    prow = recbuf[j, pl.ds(0, 16)]
                    pint = plsc.bitcast(prow, I32)
                    dl = pint[5] - lo
                    p0 = prow[0]
                    p1 = prow[1]
                    p2 = prow[2]
                    p3 = prow[3]
                    ab = dl * 128
                    for v in range(8):
                        u = (rowsbuf[j, pl.ds(v * 16, 16)]
                             + p0 * w3v[0][v] + p1 * w3v[1][v]
                             + p2 * w3v[2][v] + p3 * w3v[3][v])
                        sl = pl.ds(ab + v * 16, 16)
                        acc[sl] = jnp.maximum(acc[sl], u)
                return carry

            pltpu.make_async_copy(
                dst_hbm.at[pl.ds(0, ch)], s0, sem0).start()

            def chunk(c, n, sb, sem, sbn, semn):
                @pl.when(c + 1 < nchunk)
                def _():
                    pltpu.make_async_copy(
                        dst_hbm.at[pl.ds((c + 1) * ch, ch)], sbn,
                        semn).start()
                pltpu.make_async_copy(
                    dst_hbm.at[pl.ds(c * ch, ch)], sb, sem).wait()

                def scan_step(v, nn):
                    d = sb[pl.ds(v * 16, 16)]
                    m = plsc.bitcast(d - lo, jnp.uint32) < jnp.uint32(rng)
                    eidv = iota + (c * ch + v * 16)
                    plsc.store_compressed(eidbuf.at[pl.ds(nn, 16)], eidv,
                                          mask=m)
                    cnt = plsc.all_reduce_population_count(m)
                    return nn + cnt[0]

                n2 = lax.fori_loop(0, ch // 16, scan_step, n)
                nfull = n2 // g
                lax.fori_loop(0, nfull, process_group, 0)

                @pl.when(nfull > 0)
                def _():
                    for t in range(g // 16):
                        eidbuf[pl.ds(t * 16, 16)] = (
                            eidbuf[pl.ds(nfull * g + t * 16, 16)])
                return n2 - nfull * g

            def pair(p, n):
                n = chunk(2 * p, n, s0, sem0, s1, sem1)
                n = chunk(2 * p + 1, n, s1, sem1, s0, sem0)
                return n

            n = lax.fori_loop(0, nchunk // 2, pair, 0)

            padv = jnp.full((16,), eidbuf[pl.ds(0, 16)][0], I32)
            for t in range(g // 16):
                eidbuf[pl.ds(n + t * 16, 16)] = padv
            lax.fori_loop(0, (n + g - 1) // g, process_group, 0)

            nfl = rng // 8
            rb = [xbuf.at[pl.ds(0, 8)], xbuf.at[pl.ds(8, 8)]]
            wb = [xbuf.at[pl.ds(16, 8)], xbuf.at[pl.ds(24, 8)]]
            semr = [semr0, semr1]
            semw = [semw0, semw1]
            for h in range(2):
                pltpu.make_async_copy(
                    xw1b_hbm.at[pl.ds(lo + h * 8, 8)], rb[h],
                    semr[h]).start()

            def flush_chunk(c, h):
                row0 = lo + c * 8
                pltpu.make_async_copy(
                    xw1b_hbm.at[pl.ds(row0, 8)], rb[h], semr[h]).wait()

                @pl.when(c >= 2)
                def _():
                    pltpu.make_async_copy(
                        wb[h], out_hbm.at[pl.ds(row0 - 16, 8)],
                        semw[h]).wait()

                @pl.loop(0, 8)
                def _(rr):
                    for v in range(8):
                        s = acc[pl.ds((c * 8 + rr) * 128 + v * 16, 16)]
                        sl = pl.ds(v * 16, 16)
                        wb[h][rr, sl] = jnp.maximum(rb[h][rr, sl] + s, 0.0)

                pltpu.make_async_copy(
                    wb[h], out_hbm.at[pl.ds(row0, 8)], semw[h]).start()

                @pl.when(c + 2 < nfl)
                def _():
                    pltpu.make_async_copy(
                        xw1b_hbm.at[pl.ds(row0 + 16, 8)], rb[h],
                        semr[h]).start()

            @pl.loop(0, nfl // 2)
            def _(t):
                flush_chunk(2 * t, 0)
                flush_chunk(2 * t + 1, 1)

            for h in range(2):
                pltpu.make_async_copy(
                    wb[h], out_hbm.at[pl.ds(lo + rng - 16 + h * 8, 8)],
                    semw[h]).wait()

    return k(dst, rec, xw2, xw1b, w3flat)


def _out_gather(idxp, outfull):
    nspad = idxp.shape[0]
    nsw = nspad // NW
    g = 400
    niter = nsw // g

    @functools.partial(
        pl.kernel,
        out_type=jax.ShapeDtypeStruct((nspad, 128), F32),
        mesh=_make_mesh(),
        compiler_params=_sc_params(),
        scratch_types=[
            pltpu.VMEM((g,), I32),
            pltpu.VMEM((g, 128), F32),
        ],
    )
    def k(idx_hbm, full_hbm, o_hbm, ibuf, rbuf):
        wid = lax.axis_index("s") * NC + lax.axis_index("c")

        @pl.loop(0, niter)
        def _(it):
            off = wid * nsw + it * g
            pltpu.sync_copy(idx_hbm.at[pl.ds(off, g)], ibuf)
            pltpu.sync_copy(full_hbm.at[ibuf], rbuf)
            pltpu.sync_copy(rbuf, o_hbm.at[pl.ds(off, g)])

    return k(idxp, outfull)


def kernel(x, pos, batch, norm, edge_index, idx, W, b):
    n, d = x.shape
    e = edge_index.shape[1]
    ns = idx.shape[0]
    assert d == 128 and n == 50000 and e == 400000 and ns == 12500

    npad = 50176
    epad = 409600
    nspad = 12800

    src = edge_index[0]
    dst = edge_index[1]

    wc = jnp.concatenate([W[:d], W[d:2 * d]], axis=1)
    w3flat = W[2 * d:2 * d + 4].reshape(-1)

    batchf = lax.bitcast_convert_type(batch, F32).reshape(n, 1)
    pn = jnp.concatenate(
        [pos, norm, batchf, jnp.zeros((n, 9), F32)], axis=1)

    x_pad = jnp.concatenate([x, jnp.zeros((npad - n, d), F32)], axis=0)
    srcp = jnp.concatenate([src, jnp.zeros((epad - e,), I32)])
    dstp = jnp.concatenate([dst, jnp.zeros((epad - e,), I32)])
    idxp = jnp.concatenate([idx, jnp.zeros((nspad - ns,), I32)])

    xw1b, xw2 = _matmul(x_pad, wc, b)
    ps_ext, pd_ext, pni = _gather_rows(pn, srcp, dstp, idxp)
    rec = _ppf(ps_ext.T, pd_ext.T).T
    outfull = _segmax(dst, rec, xw2, xw1b, w3flat)
    outg = _out_gather(idxp, outfull)

    x_out = outg[:ns]
    pos_out = pni[:ns, 0:3]
    batch_out = lax.bitcast_convert_type(pni[:ns, 6], I32)
    return (x_out, pos_out, batch_out, idx)


# speedup vs baseline: 3.6801x; 1.3117x over previous
import dataclasses
import functools

import jax
import jax.numpy as jnp
from jax import lax
from jax.experimental import pallas as pl
from jax.experimental.pallas import tpu as pltpu
from jax.experimental.pallas import tpu_sc as plsc

F32 = jnp.float32
BF16 = jnp.bfloat16
I32 = jnp.int32

NW = 32
NC = 2


def _mm_body(x_ref, w_ref, b_ref, o1_ref, o2_ref):
    d = x_ref.shape[1]
    acc = jnp.dot(x_ref[...], w_ref[...], preferred_element_type=F32)
    o1_ref[...] = acc[:, :d] + b_ref[...]
    o2_ref[...] = acc[:, d:]


def _matmul(x_pad, wc, b):
    npad, d = x_pad.shape
    blk = 1024
    grid = npad // blk
    return pl.pallas_call(
        _mm_body,
        grid=(grid,),
        in_specs=[
            pl.BlockSpec((blk, d), lambda i: (i, 0)),
            pl.BlockSpec((d, 2 * d), lambda i: (0, 0)),
            pl.BlockSpec((1, d), lambda i: (0, 0)),
        ],
        out_specs=[
            pl.BlockSpec((blk, d), lambda i: (i, 0)),
            pl.BlockSpec((blk, d), lambda i: (i, 0)),
        ],
        out_shape=[
            jax.ShapeDtypeStruct((npad, d), F32),
            jax.ShapeDtypeStruct((npad, d), F32),
        ],
    )(x_pad, wc, b.reshape(1, d))


def _ppf_body(ps_ref, pd_ref, o_ref):
    ps = ps_ref[...]
    pd = pd_ref[...]

    def cross_dot(v1, v2):
        cx = v1[1:2] * v2[2:3] - v1[2:3] * v2[1:2]
        cy = v1[2:3] * v2[0:1] - v1[0:1] * v2[2:3]
        cz = v1[0:1] * v2[1:2] - v1[1:2] * v2[0:1]
        cn = jnp.sqrt(cx * cx + cy * cy + cz * cz)
        dt = v1[0:1] * v2[0:1] + v1[1:2] * v2[1:2] + v1[2:3] * v2[2:3]
        return jnp.arctan2(cn, dt)

    pos_s, n_s = ps[0:3], ps[3:6]
    pos_d, n_d = pd[0:3], pd[3:6]
    pseudo = pos_s - pos_d
    p0 = jnp.sqrt(jnp.sum(pseudo * pseudo, axis=0, keepdims=True))
    p1 = cross_dot(n_d, pseudo)
    p2 = cross_dot(n_s, pseudo)
    p3 = cross_dot(n_d, n_s)
    zero = jnp.zeros((10, ps.shape[1]), F32)
    o_ref[...] = jnp.concatenate(
        [p0, p1, p2, p3, ps[7:8], pd[7:8], zero], axis=0)


def _ppf(ps_t, pd_t):
    epad = ps_t.shape[1]
    blk = 1024
    grid = epad // blk
    return pl.pallas_call(
        _ppf_body,
        grid=(grid,),
        in_specs=[
            pl.BlockSpec((16, blk), lambda i: (0, i)),
            pl.BlockSpec((16, blk), lambda i: (0, i)),
        ],
        out_specs=pl.BlockSpec((16, blk), lambda i: (0, i)),
        out_shape=jax.ShapeDtypeStruct((16, epad), F32),
    )(ps_t, pd_t)


def _make_mesh():
    return plsc.VectorSubcoreMesh(core_axis_name="c", subcore_axis_name="s")


def _sc_params():
    cp = pltpu.CompilerParams()
    if "needs_layout_passes" in pltpu.CompilerParams.__dataclass_fields__:
        cp = dataclasses.replace(cp, needs_layout_passes=False)
    if "use_tc_tiling_on_sc" in pltpu.CompilerParams.__dataclass_fields__:
        cp = dataclasses.replace(cp, use_tc_tiling_on_sc=False)
    return cp


def _gather_rows(pn, srcp, dstp, idxp):
    epad = srcp.shape[0]
    nspad = idxp.shape[0]
    ew = epad // NW
    g = 512
    niter = ew // g
    nsw = nspad // NW
    g2 = 400
    niter2 = nsw // g2

    @functools.partial(
        pl.kernel,
        out_type=(
            jax.ShapeDtypeStruct((epad, 16), F32),
            jax.ShapeDtypeStruct((epad, 16), F32),
            jax.ShapeDtypeStruct((nspad, 16), F32),
        ),
        mesh=_make_mesh(),
        compiler_params=_sc_params(),
        scratch_types=[
            pltpu.VMEM((g,), I32),
            pltpu.VMEM((g, 16), F32),
            pltpu.VMEM((g2,), I32),
            pltpu.VMEM((g2, 16), F32),
        ],
    )
    def k(pn_hbm, src_hbm, dst_hbm, idx_hbm, ps_hbm, pd_hbm, pni_hbm,
          ibuf, rbuf, ibuf2, rbuf2):
        wid = lax.axis_index("s") * NC + lax.axis_index("c")
        iota = lax.iota(I32, 16)
        col7 = jnp.full((16,), 7, I32)

        def tagged(idx_src, out_hbm):
            base = wid * ew

            @pl.loop(0, niter)
            def _(it):
                off = base + it * g
                pltpu.sync_copy(idx_src.at[pl.ds(off, g)], ibuf)
                pltpu.sync_copy(pn_hbm.at[ibuf], rbuf)
                for v in range(g // 16):
                    rows = iota + v * 16
                    vals = plsc.bitcast(ibuf[pl.ds(v * 16, 16)], F32)
                    plsc.store_scatter(rbuf, [rows, col7], vals)
                pltpu.sync_copy(rbuf, out_hbm.at[pl.ds(off, g)])

        tagged(src_hbm, ps_hbm)
        tagged(dst_hbm, pd_hbm)

        base2 = wid * nsw

        @pl.loop(0, niter2)
        def _(it):
            off = base2 + it * g2
            pltpu.sync_copy(idx_hbm.at[pl.ds(off, g2)], ibuf2)
            pltpu.sync_copy(pn_hbm.at[ibuf2], rbuf2)
            pltpu.sync_copy(rbuf2, pni_hbm.at[pl.ds(off, g2)])

    return k(pn, srcp, dstp, idxp)


def _segmax(dst, rec, xw2, xw1b, w3flat):
    e = dst.shape[0]
    npad = xw1b.shape[0]
    rng = 1568
    nrounds = npad // (rng * NW)
    ch = 1600
    nchunk = e // ch
    g = 128

    @functools.partial(
        pl.kernel,
        out_type=jax.ShapeDtypeStruct((npad, 128), F32),
        mesh=_make_mesh(),
        compiler_params=_sc_params(),
        scratch_types=[
            pltpu.VMEM((rng * 128,), BF16),
            pltpu.VMEM((ch,), I32),
            pltpu.VMEM((ch,), I32),
            pltpu.VMEM((1856,), I32),
            pltpu.VMEM((g, 16), F32),
            pltpu.VMEM((g,), I32),
            pltpu.VMEM((g, 128), F32),
            pltpu.VMEM((512,), F32),
            pltpu.VMEM((32, 128), F32),
            pltpu.SemaphoreType.DMA,
            pltpu.SemaphoreType.DMA,
            pltpu.SemaphoreType.DMA,
            pltpu.SemaphoreType.DMA,
            pltpu.SemaphoreType.DMA,
            pltpu.SemaphoreType.DMA,
        ],
    )
    def k(dst_hbm, rec_hbm, xw2_hbm, xw1b_hbm, w3_hbm, out_hbm,
          acc, s0, s1, eidbuf, recbuf, srcbuf, rowsbuf, w3buf, xbuf,
          sem0, sem1, semr0, semr1, semw0, semw1):
        wid = lax.axis_index("s") * NC + lax.axis_index("c")
        pltpu.sync_copy(w3_hbm, w3buf)
        w3v = [[w3buf[pl.ds(r * 128 + v * 16, 16)] for v in range(8)]
               for r in range(4)]
        iota = lax.iota(I32, 16)
        col4 = jnp.full((16,), 4, I32)
        neg = jnp.full((16,), -3e38, F32)

        @pl.loop(0, nrounds)
        def _(rnd):
            lo = (rnd * NW + wid) * rng

            negb = plsc.pack(neg, neg, format=plsc.PackFormat.INTERLEAVED)

            @pl.loop(0, rng * 128 // 32)
            def _(i):
                acc[pl.ds(i * 32, 32)] = negb

            def process_group(i, carry):
                pltpu.sync_copy(
                    rec_hbm.at[eidbuf.at[pl.ds(i * g, g)]], recbuf)
                for v in range(g // 16):
                    sf = plsc.load_gather(recbuf, [iota + v * 16, col4])
                    srcbuf[pl.ds(v * 16, 16)] = plsc.bitcast(sf, I32)
                pltpu.sync_copy(xw2_hbm.at[srcbuf], rowsbuf)

                @pl.loop(0, g)
                def _(j):
                    prow = recbuf[j, pl.ds(0, 16)]
                    pint = plsc.bitcast(prow, I32)
                    dl = pint[5] - lo
                    p0 = prow[0]
                    p1 = prow[1]
                    p2 = prow[2]
                    p3 = prow[3]
                    ab = dl * 128
                    for v in range(4):
                        u0 = (rowsbuf[j, pl.ds(v * 32, 16)]
                              + p0 * w3v[0][2 * v] + p1 * w3v[1][2 * v]
                              + p2 * w3v[2][2 * v] + p3 * w3v[3][2 * v])
                        u1 = (rowsbuf[j, pl.ds(v * 32 + 16, 16)]
                              + p0 * w3v[0][2 * v + 1]
                              + p1 * w3v[1][2 * v + 1]
                              + p2 * w3v[2][2 * v + 1]
                              + p3 * w3v[3][2 * v + 1])
                        sl = pl.ds(ab + v * 32, 32)
                        a0, a1 = plsc.unpack(
                            acc[sl], format=plsc.PackFormat.INTERLEAVED)
                        acc[sl] = plsc.pack(
                            jnp.maximum(a0, u0), jnp.maximum(a1, u1),
                            format=plsc.PackFormat.INTERLEAVED)
                return carry

            pltpu.make_async_copy(
                dst_hbm.at[pl.ds(0, ch)], s0, sem0).start()

            def chunk(c, n, sb, sem, sbn, semn):
                @pl.when(c + 1 < nchunk)
                def _():
                    pltpu.make_async_copy(
                        dst_hbm.at[pl.ds((c + 1) * ch, ch)], sbn,
                        semn).start()
                pltpu.make_async_copy(
                    dst_hbm.at[pl.ds(c * ch, ch)], sb, sem).wait()

                def scan_step(v, nn):
                    d = sb[pl.ds(v * 16, 16)]
                    m = plsc.bitcast(d - lo, jnp.uint32) < jnp.uint32(rng)
                    eidv = iota + (c * ch + v * 16)
                    plsc.store_compressed(eidbuf.at[pl.ds(nn, 16)], eidv,
                                          mask=m)
                    cnt = plsc.all_reduce_population_count(m)
                    return nn + cnt[0]

                n2 = lax.fori_loop(0, ch // 16, scan_step, n)
                nfull = n2 // g
                lax.fori_loop(0, nfull, process_group, 0)

                @pl.when(nfull > 0)
                def _():
                    for t in range(g // 16):
                        eidbuf[pl.ds(t * 16, 16)] = (
                            eidbuf[pl.ds(nfull * g + t * 16, 16)])
                return n2 - nfull * g

            def pair(p, n):
                n = chunk(2 * p, n, s0, sem0, s1, sem1)
                n = chunk(2 * p + 1, n, s1, sem1, s0, sem0)
                return n

            n = lax.fori_loop(0, nchunk // 2, pair, 0)

            padv = jnp.full((16,), eidbuf[pl.ds(0, 16)][0], I32)
            for t in range(g // 16):
                eidbuf[pl.ds(n + t * 16, 16)] = padv
            lax.fori_loop(0, (n + g - 1) // g, process_group, 0)

            nfl = rng // 8
            rb = [xbuf.at[pl.ds(0, 8)], xbuf.at[pl.ds(8, 8)]]
            wb = [xbuf.at[pl.ds(16, 8)], xbuf.at[pl.ds(24, 8)]]
            semr = [semr0, semr1]
            semw = [semw0, semw1]
            for h in range(2):
                pltpu.make_async_copy(
                    xw1b_hbm.at[pl.ds(lo + h * 8, 8)], rb[h],
                    semr[h]).start()

            def flush_chunk(c, h):
                row0 = lo + c * 8
                pltpu.make_async_copy(
                    xw1b_hbm.at[pl.ds(row0, 8)], rb[h], semr[h]).wait()

                @pl.when(c >= 2)
                def _():
                    pltpu.make_async_copy(
                        wb[h], out_hbm.at[pl.ds(row0 - 16, 8)],
                        semw[h]).wait()

                @pl.loop(0, 8)
                def _(rr):
                    for v in range(4):
                        a0, a1 = plsc.unpack(
                            acc[pl.ds((c * 8 + rr) * 128 + v * 32, 32)],
                            format=plsc.PackFormat.INTERLEAVED)
                        sl0 = pl.ds(v * 32, 16)
                        sl1 = pl.ds(v * 32 + 16, 16)
                        wb[h][rr, sl0] = jnp.maximum(rb[h][rr, sl0] + a0,
                                                     0.0)
                        wb[h][rr, sl1] = jnp.maximum(rb[h][rr, sl1] + a1,
                                                     0.0)

                pltpu.make_async_copy(
                    wb[h], out_hbm.at[pl.ds(row0, 8)], semw[h]).start()

                @pl.when(c + 2 < nfl)
                def _():
                    pltpu.make_async_copy(
                        xw1b_hbm.at[pl.ds(row0 + 16, 8)], rb[h],
                        semr[h]).start()

            @pl.loop(0, nfl // 2)
            def _(t):
                flush_chunk(2 * t, 0)
                flush_chunk(2 * t + 1, 1)

            for h in range(2):
                pltpu.make_async_copy(
                    wb[h], out_hbm.at[pl.ds(lo + rng - 16 + h * 8, 8)],
                    semw[h]).wait()

    return k(dst, rec, xw2, xw1b, w3flat)


def _out_gather(idxp, outfull):
    nspad = idxp.shape[0]
    nsw = nspad // NW
    g = 400
    niter = nsw // g

    @functools.partial(
        pl.kernel,
        out_type=jax.ShapeDtypeStruct((nspad, 128), F32),
        mesh=_make_mesh(),
        compiler_params=_sc_params(),
        scratch_types=[
            pltpu.VMEM((g,), I32),
            pltpu.VMEM((g, 128), F32),
        ],
    )
    def k(idx_hbm, full_hbm, o_hbm, ibuf, rbuf):
        wid = lax.axis_index("s") * NC + lax.axis_index("c")

        @pl.loop(0, niter)
        def _(it):
            off = wid * nsw + it * g
            pltpu.sync_copy(idx_hbm.at[pl.ds(off, g)], ibuf)
            pltpu.sync_copy(full_hbm.at[ibuf], rbuf)
            pltpu.sync_copy(rbuf, o_hbm.at[pl.ds(off, g)])

    return k(idxp, outfull)


def kernel(x, pos, batch, norm, edge_index, idx, W, b):
    n, d = x.shape
    e = edge_index.shape[1]
    ns = idx.shape[0]
    assert d == 128 and n == 50000 and e == 400000 and ns == 12500

    npad = 50176
    epad = 409600
    nspad = 12800

    src = edge_index[0]
    dst = edge_index[1]

    wc = jnp.concatenate([W[:d], W[d:2 * d]], axis=1)
    w3flat = W[2 * d:2 * d + 4].reshape(-1)

    batchf = lax.bitcast_convert_type(batch, F32).reshape(n, 1)
    pn = jnp.concatenate(
        [pos, norm, batchf, jnp.zeros((n, 9), F32)], axis=1)

    x_pad = jnp.concatenate([x, jnp.zeros((npad - n, d), F32)], axis=0)
    srcp = jnp.concatenate([src, jnp.zeros((epad - e,), I32)])
    dstp = jnp.concatenate([dst, jnp.zeros((epad - e,), I32)])
    idxp = jnp.concatenate([idx, jnp.zeros((nspad - ns,), I32)])

    xw1b, xw2 = _matmul(x_pad, wc, b)
    ps_ext, pd_ext, pni = _gather_rows(pn, srcp, dstp, idxp)
    rec = _ppf(ps_ext.T, pd_ext.T).T
    outfull = _segmax(dst, rec, xw2, xw1b, w3flat)
    outg = _out_gather(idxp, outfull)

    x_out = outg[:ns]
    pos_out = pni[:ns, 0:3]
    batch_out = lax.bitcast_convert_type(pni[:ns, 6], I32)
    return (x_out, pos_out, batch_out, idx)


# speedup vs baseline: 3.7336x; 1.0145x over previous
import dataclasses
import functools

import jax
import jax.numpy as jnp
from jax import lax
from jax.experimental import pallas as pl
from jax.experimental.pallas import tpu as pltpu
from jax.experimental.pallas import tpu_sc as plsc

F32 = jnp.float32
BF16 = jnp.bfloat16
I32 = jnp.int32

NW = 32
NC = 2


def _mm_body(x_ref, w_ref, b_ref, o1_ref, o2_ref):
    d = x_ref.shape[1]
    acc = jnp.dot(x_ref[...], w_ref[...], preferred_element_type=F32)
    o1_ref[...] = acc[:, :d] + b_ref[...]
    o2_ref[...] = acc[:, d:]


def _matmul(x_pad, wc, b):
    npad, d = x_pad.shape
    blk = 1024
    grid = npad // blk
    return pl.pallas_call(
        _mm_body,
        grid=(grid,),
        in_specs=[
            pl.BlockSpec((blk, d), lambda i: (i, 0)),
            pl.BlockSpec((d, 2 * d), lambda i: (0, 0)),
            pl.BlockSpec((1, d), lambda i: (0, 0)),
        ],
        out_specs=[
            pl.BlockSpec((blk, d), lambda i: (i, 0)),
            pl.BlockSpec((blk, d), lambda i: (i, 0)),
        ],
        out_shape=[
            jax.ShapeDtypeStruct((npad, d), F32),
            jax.ShapeDtypeStruct((npad, d), F32),
        ],
    )(x_pad, wc, b.reshape(1, d))


def _ppf_body(ps_ref, pd_ref, o_ref):
    ps = ps_ref[...]
    pd = pd_ref[...]

    def cross_dot(v1, v2):
        cx = v1[1:2] * v2[2:3] - v1[2:3] * v2[1:2]
        cy = v1[2:3] * v2[0:1] - v1[0:1] * v2[2:3]
        cz = v1[0:1] * v2[1:2] - v1[1:2] * v2[0:1]
        cn = jnp.sqrt(cx * cx + cy * cy + cz * cz)
        dt = v1[0:1] * v2[0:1] + v1[1:2] * v2[1:2] + v1[2:3] * v2[2:3]
        return jnp.arctan2(cn, dt)

    pos_s, n_s = ps[0:3], ps[3:6]
    pos_d, n_d = pd[0:3], pd[3:6]
    pseudo = pos_s - pos_d
    p0 = jnp.sqrt(jnp.sum(pseudo * pseudo, axis=0, keepdims=True))
    p1 = cross_dot(n_d, pseudo)
    p2 = cross_dot(n_s, pseudo)
    p3 = cross_dot(n_d, n_s)
    zero = jnp.zeros((10, ps.shape[1]), F32)
    o_ref[...] = jnp.concatenate(
        [p0, p1, p2, p3, ps[7:8], pd[7:8], zero], axis=0)


def _ppf(ps_t, pd_t):
    epad = ps_t.shape[1]
    blk = 1024
    grid = epad // blk
    return pl.pallas_call(
        _ppf_body,
        grid=(grid,),
        in_specs=[
            pl.BlockSpec((16, blk), lambda i: (0, i)),
            pl.BlockSpec((16, blk), lambda i: (0, i)),
        ],
        out_specs=pl.BlockSpec((16, blk), lambda i: (0, i)),
        out_shape=jax.ShapeDtypeStruct((16, epad), F32),
    )(ps_t, pd_t)


def _make_mesh():
    return plsc.VectorSubcoreMesh(core_axis_name="c", subcore_axis_name="s")


def _sc_params():
    cp = pltpu.CompilerParams()
    if "needs_layout_passes" in pltpu.CompilerParams.__dataclass_fields__:
        cp = dataclasses.replace(cp, needs_layout_passes=False)
    if "use_tc_tiling_on_sc" in pltpu.CompilerParams.__dataclass_fields__:
        cp = dataclasses.replace(cp, use_tc_tiling_on_sc=False)
    return cp


def _gather_rows(pn, srcp, dstp, idxp):
    epad = srcp.shape[0]
    nspad = idxp.shape[0]
    ew = epad // NW
    g = 512
    niter = ew // g
    nsw = nspad // NW
    g2 = 400
    niter2 = nsw // g2

    @functools.partial(
        pl.kernel,
        out_type=(
            jax.ShapeDtypeStruct((epad, 16), F32),
            jax.ShapeDtypeStruct((epad, 16), F32),
            jax.ShapeDtypeStruct((nspad, 16), F32),
        ),
        mesh=_make_mesh(),
        compiler_params=_sc_params(),
        scratch_types=[
            pltpu.VMEM((g,), I32),
            pltpu.VMEM((g, 16), F32),
            pltpu.VMEM((g2,), I32),
            pltpu.VMEM((g2, 16), F32),
        ],
    )
    def k(pn_hbm, src_hbm, dst_hbm, idx_hbm, ps_hbm, pd_hbm, pni_hbm,
          ibuf, rbuf, ibuf2, rbuf2):
        wid = lax.axis_index("s") * NC + lax.axis_index("c")
        iota = lax.iota(I32, 16)
        col7 = jnp.full((16,), 7, I32)

        def tagged(idx_src, out_hbm):
            base = wid * ew

            @pl.loop(0, niter)
            def _(it):
                off = base + it * g
                pltpu.sync_copy(idx_src.at[pl.ds(off, g)], ibuf)
                pltpu.sync_copy(pn_hbm.at[ibuf], rbuf)
                for v in range(g // 16):
                    rows = iota + v * 16
                    vals = plsc.bitcast(ibuf[pl.ds(v * 16, 16)], F32)
                    plsc.store_scatter(rbuf, [rows, col7], vals)
                pltpu.sync_copy(rbuf, out_hbm.at[pl.ds(off, g)])

        tagged(src_hbm, ps_hbm)
        tagged(dst_hbm, pd_hbm)

        base2 = wid * nsw

        @pl.loop(0, niter2)
        def _(it):
            off = base2 + it * g2
            pltpu.sync_copy(idx_hbm.at[pl.ds(off, g2)], ibuf2)
            pltpu.sync_copy(pn_hbm.at[ibuf2], rbuf2)
            pltpu.sync_copy(rbuf2, pni_hbm.at[pl.ds(off, g2)])

    return k(pn, srcp, dstp, idxp)


def _segmax(dst, rec, xw2, xw1b, w3flat):
    e = dst.shape[0]
    npad = xw1b.shape[0]
    rng = 1568
    nrounds = npad // (rng * NW)
    ch = 1600
    nchunk = e // ch
    g = 64

    @functools.partial(
        pl.kernel,
        out_type=jax.ShapeDtypeStruct((npad, 128), F32),
        mesh=_make_mesh(),
        compiler_params=_sc_params(),
        scratch_types=[
            pltpu.VMEM((rng * 128,), BF16),
            pltpu.VMEM((ch,), I32),
            pltpu.VMEM((ch,), I32),
            pltpu.VMEM((1856,), I32),
            pltpu.VMEM((g, 16), F32),
            pltpu.VMEM((g, 16), F32),
            pltpu.VMEM((g,), I32),
            pltpu.VMEM((g,), I32),
            pltpu.VMEM((g, 128), F32),
            pltpu.VMEM((g, 128), F32),
            pltpu.VMEM((512,), F32),
            pltpu.VMEM((32, 128), F32),
            pltpu.SemaphoreType.DMA,
            pltpu.SemaphoreType.DMA,
            pltpu.SemaphoreType.DMA,
            pltpu.SemaphoreType.DMA,
            pltpu.SemaphoreType.DMA,
            pltpu.SemaphoreType.DMA,
            pltpu.SemaphoreType.DMA,
            pltpu.SemaphoreType.DMA,
            pltpu.SemaphoreType.DMA,
            pltpu.SemaphoreType.DMA,
        ],
    )
    def k(dst_hbm, rec_hbm, xw2_hbm, xw1b_hbm, w3_hbm, out_hbm,
          acc, s0, s1, eidbuf, recbuf, recbuf2, srcbuf, srcbuf2,
          rowsbuf, rowsbuf2, w3buf, xbuf,
          sem0, sem1, semr0, semr1, semw0, semw1, semga, semgb,
          semxa, semxb):
        wid = lax.axis_index("s") * NC + lax.axis_index("c")
        pltpu.sync_copy(w3_hbm, w3buf)
        w3v = [[w3buf[pl.ds(r * 128 + v * 16, 16)] for v in range(8)]
               for r in range(4)]
        iota = lax.iota(I32, 16)
        col4 = jnp.full((16,), 4, I32)
        neg = jnp.full((16,), -3e38, F32)

        @pl.loop(0, nrounds)
        def _(rnd):
            lo = (rnd * NW + wid) * rng

            negb = plsc.pack(neg, neg, format=plsc.PackFormat.INTERLEAVED)

            @pl.loop(0, rng * 128 // 32)
            def _(i):
                acc[pl.ds(i * 32, 32)] = negb

            def extract_src(recb, srcb):
                for v in range(g // 16):
                    sf = plsc.load_gather(recb, [iota + v * 16, col4])
                    srcb[pl.ds(v * 16, 16)] = plsc.bitcast(sf, I32)

            def edge_loop(recb, rowsb):
                @pl.loop(0, g)
                def _(j):
                    prow = recb[j, pl.ds(0, 16)]
                    pint = plsc.bitcast(prow, I32)
                    dl = pint[5] - lo
                    p0 = prow[0]
                    p1 = prow[1]
                    p2 = prow[2]
                    p3 = prow[3]
                    ab = dl * 128
                    for v in range(4):
                        u0 = (rowsb[j, pl.ds(v * 32, 16)]
                              + p0 * w3v[0][2 * v] + p1 * w3v[1][2 * v]
                              + p2 * w3v[2][2 * v] + p3 * w3v[3][2 * v])
                        u1 = (rowsb[j, pl.ds(v * 32 + 16, 16)]
                              + p0 * w3v[0][2 * v + 1]
                              + p1 * w3v[1][2 * v + 1]
                              + p2 * w3v[2][2 * v + 1]
                              + p3 * w3v[3][2 * v + 1])
                        sl = pl.ds(ab + v * 32, 32)
                        a0, a1 = plsc.unpack(
                            acc[sl], format=plsc.PackFormat.INTERLEAVED)
                        acc[sl] = plsc.pack(
                            jnp.maximum(a0, u0), jnp.maximum(a1, u1),
                            format=plsc.PackFormat.INTERLEAVED)

            def process_group(i, carry):
                pltpu.sync_copy(
                    rec_hbm.at[eidbuf.at[pl.ds(i * g, g)]], recbuf)
                extract_src(recbuf, srcbuf)
                pltpu.sync_copy(xw2_hbm.at[srcbuf], rowsbuf)
                edge_loop(recbuf, rowsbuf)
                return carry

            def process_pair(i, carry):
                offa = i * 2 * g
                offb = offa + g
                ga = rec_hbm.at[eidbuf.at[pl.ds(offa, g)]]
                gb = rec_hbm.at[eidbuf.at[pl.ds(offb, g)]]
                pltpu.make_async_copy(ga, recbuf, semga).start()
                pltpu.make_async_copy(gb, recbuf2, semgb).start()
                pltpu.make_async_copy(ga, recbuf, semga).wait()
                extract_src(recbuf, srcbuf)
                pltpu.make_async_copy(
                    xw2_hbm.at[srcbuf], rowsbuf, semxa).start()
                pltpu.make_async_copy(gb, recbuf2, semgb).wait()
                extract_src(recbuf2, srcbuf2)
                pltpu.make_async_copy(
                    xw2_hbm.at[srcbuf2], rowsbuf2, semxb).start()
                pltpu.make_async_copy(
                    xw2_hbm.at[srcbuf], rowsbuf, semxa).wait()
                edge_loop(recbuf, rowsbuf)
                pltpu.make_async_copy(
                    xw2_hbm.at[srcbuf2], rowsbuf2, semxb).wait()
                edge_loop(recbuf2, rowsbuf2)
                return carry

            pltpu.make_async_copy(
                dst_hbm.at[pl.ds(0, ch)], s0, sem0).start()

            def chunk(c, n, sb, sem, sbn, semn):
                @pl.when(c + 1 < nchunk)
                def _():
                    pltpu.make_async_copy(
                        dst_hbm.at[pl.ds((c + 1) * ch, ch)], sbn,
                        semn).start()
                pltpu.make_async_copy(
                    dst_hbm.at[pl.ds(c * ch, ch)], sb, sem).wait()

                def scan_step(v, nn):
                    d = sb[pl.ds(v * 16, 16)]
                    m = plsc.bitcast(d - lo, jnp.uint32) < jnp.uint32(rng)
                    eidv = iota + (c * ch + v * 16)
                    plsc.store_compressed(eidbuf.at[pl.ds(nn, 16)], eidv,
                                          mask=m)
                    cnt = plsc.all_reduce_population_count(m)
                    return nn + cnt[0]

                n2 = lax.fori_loop(0, ch // 16, scan_step, n)
                npairs = n2 // (2 * g)
                lax.fori_loop(0, npairs, process_pair, 0)

                @pl.when(npairs > 0)
                def _():
                    for t in range(2 * g // 16):
                        eidbuf[pl.ds(t * 16, 16)] = (
                            eidbuf[pl.ds(npairs * 2 * g + t * 16, 16)])
                return n2 - npairs * 2 * g

            def pair(p, n):
                n = chunk(2 * p, n, s0, sem0, s1, sem1)
                n = chunk(2 * p + 1, n, s1, sem1, s0, sem0)
                return n

            n = lax.fori_loop(0, nchunk // 2, pair, 0)

            padv = jnp.full((16,), eidbuf[pl.ds(0, 16)][0], I32)
            for t in range(g // 16):
                eidbuf[pl.ds(n + t * 16, 16)] = padv
            lax.fori_loop(0, (n + g - 1) // g, process_group, 0)

            nfl = rng // 8
            rb = [xbuf.at[pl.ds(0, 8)], xbuf.at[pl.ds(8, 8)]]
            wb = [xbuf.at[pl.ds(16, 8)], xbuf.at[pl.ds(24, 8)]]
            semr = [semr0, semr1]
            semw = [semw0, semw1]
            for h in range(2):
                pltpu.make_async_copy(
                    xw1b_hbm.at[pl.ds(lo + h * 8, 8)], rb[h],
                    semr[h]).start()

            def flush_chunk(c, h):
                row0 = lo + c * 8
                pltpu.make_async_copy(
                    xw1b_hbm.at[pl.ds(row0, 8)], rb[h], semr[h]).wait()

                @pl.when(c >= 2)
                def _():
                    pltpu.make_async_copy(
                        wb[h], out_hbm.at[pl.ds(row0 - 16, 8)],
                        semw[h]).wait()

                @pl.loop(0, 8)
                def _(rr):
                    for v in range(4):
                        a0, a1 = plsc.unpack(
                            acc[pl.ds((c * 8 + rr) * 128 + v * 32, 32)],
                            format=plsc.PackFormat.INTERLEAVED)
                        sl0 = pl.ds(v * 32, 16)
                        sl1 = pl.ds(v * 32 + 16, 16)
                        wb[h][rr, sl0] = jnp.maximum(rb[h][rr, sl0] + a0,
                                                     0.0)
                        wb[h][rr, sl1] = jnp.maximum(rb[h][rr, sl1] + a1,
                                                     0.0)

                pltpu.make_async_copy(
                    wb[h], out_hbm.at[pl.ds(row0, 8)], semw[h]).start()

                @pl.when(c + 2 < nfl)
                def _():
                    pltpu.make_async_copy(
                        xw1b_hbm.at[pl.ds(row0 + 16, 8)], rb[h],
                        semr[h]).start()

            @pl.loop(0, nfl // 2)
            def _(t):
                flush_chunk(2 * t, 0)
                flush_chunk(2 * t + 1, 1)

            for h in range(2):
                pltpu.make_async_copy(
                    wb[h], out_hbm.at[pl.ds(lo + rng - 16 + h * 8, 8)],
                    semw[h]).wait()

    return k(dst, rec, xw2, xw1b, w3flat)


def _out_gather(idxp, outfull):
    nspad = idxp.shape[0]
    nsw = nspad // NW
    g = 400
    niter = nsw // g

    @functools.partial(
        pl.kernel,
        out_type=jax.ShapeDtypeStruct((nspad, 128), F32),
        mesh=_make_mesh(),
        compiler_params=_sc_params(),
        scratch_types=[
            pltpu.VMEM((g,), I32),
            pltpu.VMEM((g, 128), F32),
        ],
    )
    def k(idx_hbm, full_hbm, o_hbm, ibuf, rbuf):
        wid = lax.axis_index("s") * NC + lax.axis_index("c")

        @pl.loop(0, niter)
        def _(it):
            off = wid * nsw + it * g
            pltpu.sync_copy(idx_hbm.at[pl.ds(off, g)], ibuf)
            pltpu.sync_copy(full_hbm.at[ibuf], rbuf)
            pltpu.sync_copy(rbuf, o_hbm.at[pl.ds(off, g)])

    return k(idxp, outfull)


def kernel(x, pos, batch, norm, edge_index, idx, W, b):
    n, d = x.shape
    e = edge_index.shape[1]
    ns = idx.shape[0]
    assert d == 128 and n == 50000 and e == 400000 and ns == 12500

    npad = 50176
    epad = 409600
    nspad = 12800

    src = edge_index[0]
    dst = edge_index[1]

    wc = jnp.concatenate([W[:d], W[d:2 * d]], axis=1)
    w3flat = W[2 * d:2 * d + 4].reshape(-1)

    batchf = lax.bitcast_convert_type(batch, F32).reshape(n, 1)
    pn = jnp.concatenate(
        [pos, norm, batchf, jnp.zeros((n, 9), F32)], axis=1)

    x_pad = jnp.concatenate([x, jnp.zeros((npad - n, d), F32)], axis=0)
    srcp = jnp.concatenate([src, jnp.zeros((epad - e,), I32)])
    dstp = jnp.concatenate([dst, jnp.zeros((epad - e,), I32)])
    idxp = jnp.concatenate([idx, jnp.zeros((nspad - ns,), I32)])

    xw1b, xw2 = _matmul(x_pad, wc, b)
    ps_ext, pd_ext, pni = _gather_rows(pn, srcp, dstp, idxp)
    rec = _ppf(ps_ext.T, pd_ext.T).T
    outfull = _segmax(dst, rec, xw2, xw1b, w3flat)
    outg = _out_gather(idxp, outfull)

    x_out = outg[:ns]
    pos_out = pni[:ns, 0:3]
    batch_out = lax.bitcast_convert_type(pni[:ns, 6], I32)
    return (x_out, pos_out, batch_out, idx)


# speedup vs baseline: 3.8462x; 1.0301x over previous
import dataclasses
import functools

import jax
import jax.numpy as jnp
from jax import lax
from jax.experimental import pallas as pl
from jax.experimental.pallas import tpu as pltpu
from jax.experimental.pallas import tpu_sc as plsc

F32 = jnp.float32
BF16 = jnp.bfloat16
I32 = jnp.int32

NW = 32
NC = 2


def _mm_body(x_ref, w_ref, b_ref, o1_ref, o2_ref):
    d = x_ref.shape[1]
    acc = jnp.dot(x_ref[...], w_ref[...], preferred_element_type=F32)
    o1_ref[...] = acc[:, :d] + b_ref[...]
    o2_ref[...] = acc[:, d:].astype(BF16)


def _matmul(x_pad, wc, b):
    npad, d = x_pad.shape
    blk = 1024
    grid = npad // blk
    return pl.pallas_call(
        _mm_body,
        grid=(grid,),
        in_specs=[
            pl.BlockSpec((blk, d), lambda i: (i, 0)),
            pl.BlockSpec((d, 2 * d), lambda i: (0, 0)),
            pl.BlockSpec((1, d), lambda i: (0, 0)),
        ],
        out_specs=[
            pl.BlockSpec((blk, d), lambda i: (i, 0)),
            pl.BlockSpec((blk, d), lambda i: (i, 0)),
        ],
        out_shape=[
            jax.ShapeDtypeStruct((npad, d), F32),
            jax.ShapeDtypeStruct((npad, d), BF16),
        ],
    )(x_pad, wc, b.reshape(1, d))


def _ppf_body(ps_ref, pd_ref, o_ref):
    ps = ps_ref[...]
    pd = pd_ref[...]

    def cross_dot(v1, v2):
        cx = v1[1:2] * v2[2:3] - v1[2:3] * v2[1:2]
        cy = v1[2:3] * v2[0:1] - v1[0:1] * v2[2:3]
        cz = v1[0:1] * v2[1:2] - v1[1:2] * v2[0:1]
        cn = jnp.sqrt(cx * cx + cy * cy + cz * cz)
        dt = v1[0:1] * v2[0:1] + v1[1:2] * v2[1:2] + v1[2:3] * v2[2:3]
        return jnp.arctan2(cn, dt)

    pos_s, n_s = ps[0:3], ps[3:6]
    pos_d, n_d = pd[0:3], pd[3:6]
    pseudo = pos_s - pos_d
    p0 = jnp.sqrt(jnp.sum(pseudo * pseudo, axis=0, keepdims=True))
    p1 = cross_dot(n_d, pseudo)
    p2 = cross_dot(n_s, pseudo)
    p3 = cross_dot(n_d, n_s)
    zero = jnp.zeros((10, ps.shape[1]), F32)
    o_ref[...] = jnp.concatenate(
        [p0, p1, p2, p3, ps[7:8], pd[7:8], zero], axis=0)


def _ppf(ps_t, pd_t):
    epad = ps_t.shape[1]
    blk = 1024
    grid = epad // blk
    return pl.pallas_call(
        _ppf_body,
        grid=(grid,),
        in_specs=[
            pl.BlockSpec((16, blk), lambda i: (0, i)),
            pl.BlockSpec((16, blk), lambda i: (0, i)),
        ],
        out_specs=pl.BlockSpec((16, blk), lambda i: (0, i)),
        out_shape=jax.ShapeDtypeStruct((16, epad), F32),
    )(ps_t, pd_t)


def _make_mesh():
    return plsc.VectorSubcoreMesh(core_axis_name="c", subcore_axis_name="s")


def _sc_params():
    cp = pltpu.CompilerParams()
    if "needs_layout_passes" in pltpu.CompilerParams.__dataclass_fields__:
        cp = dataclasses.replace(cp, needs_layout_passes=False)
    if "use_tc_tiling_on_sc" in pltpu.CompilerParams.__dataclass_fields__:
        cp = dataclasses.replace(cp, use_tc_tiling_on_sc=False)
    return cp


def _gather_rows(pn, srcp, dstp, idxp):
    epad = srcp.shape[0]
    nspad = idxp.shape[0]
    ew = epad // NW
    g = 512
    niter = ew // g
    nsw = nspad // NW
    g2 = 400
    niter2 = nsw // g2

    @functools.partial(
        pl.kernel,
        out_type=(
            jax.ShapeDtypeStruct((epad, 16), F32),
            jax.ShapeDtypeStruct((epad, 16), F32),
            jax.ShapeDtypeStruct((nspad, 16), F32),
        ),
        mesh=_make_mesh(),
        compiler_params=_sc_params(),
        scratch_types=[
            pltpu.VMEM((g,), I32),
            pltpu.VMEM((g, 16), F32),
            pltpu.VMEM((g2,), I32),
            pltpu.VMEM((g2, 16), F32),
        ],
    )
    def k(pn_hbm, src_hbm, dst_hbm, idx_hbm, ps_hbm, pd_hbm, pni_hbm,
          ibuf, rbuf, ibuf2, rbuf2):
        wid = lax.axis_index("s") * NC + lax.axis_index("c")
        iota = lax.iota(I32, 16)
        col7 = jnp.full((16,), 7, I32)

        def tagged(idx_src, out_hbm):
            base = wid * ew

            @pl.loop(0, niter)
            def _(it):
                off = base + it * g
                pltpu.sync_copy(idx_src.at[pl.ds(off, g)], ibuf)
                pltpu.sync_copy(pn_hbm.at[ibuf], rbuf)
                for v in range(g // 16):
                    rows = iota + v * 16
                    vals = plsc.bitcast(ibuf[pl.ds(v * 16, 16)], F32)
                    plsc.store_scatter(rbuf, [rows, col7], vals)
                pltpu.sync_copy(rbuf, out_hbm.at[pl.ds(off, g)])

        tagged(src_hbm, ps_hbm)
        tagged(dst_hbm, pd_hbm)

        base2 = wid * nsw

        @pl.loop(0, niter2)
        def _(it):
            off = base2 + it * g2
            pltpu.sync_copy(idx_hbm.at[pl.ds(off, g2)], ibuf2)
            pltpu.sync_copy(pn_hbm.at[ibuf2], rbuf2)
            pltpu.sync_copy(rbuf2, pni_hbm.at[pl.ds(off, g2)])

    return k(pn, srcp, dstp, idxp)


def _segmax(dst, rec, xw2, xw1b, w3flat):
    e = dst.shape[0]
    npad = xw1b.shape[0]
    rng = 1568
    nrounds = npad // (rng * NW)
    ch = 1600
    nchunk = e // ch
    g = 64

    @functools.partial(
        pl.kernel,
        out_type=jax.ShapeDtypeStruct((npad, 128), F32),
        mesh=_make_mesh(),
        compiler_params=_sc_params(),
        scratch_types=[
            pltpu.VMEM((rng * 128,), BF16),
            pltpu.VMEM((ch,), I32),
            pltpu.VMEM((ch,), I32),
            pltpu.VMEM((2048,), I32),
            pltpu.VMEM((g, 16), F32),
            pltpu.VMEM((g, 16), F32),
            pltpu.VMEM((g, 16), F32),
            pltpu.VMEM((g, 16), F32),
            pltpu.VMEM((g,), I32),
            pltpu.VMEM((g,), I32),
            pltpu.VMEM((g,), I32),
            pltpu.VMEM((g,), I32),
            pltpu.VMEM((g, 128), BF16),
            pltpu.VMEM((g, 128), BF16),
            pltpu.VMEM((g, 128), BF16),
            pltpu.VMEM((g, 128), BF16),
            pltpu.VMEM((512,), F32),
            pltpu.VMEM((32, 128), F32),
            pltpu.SemaphoreType.DMA,
            pltpu.SemaphoreType.DMA,
            pltpu.SemaphoreType.DMA,
            pltpu.SemaphoreType.DMA,
            pltpu.SemaphoreType.DMA,
            pltpu.SemaphoreType.DMA,
            pltpu.SemaphoreType.DMA,
            pltpu.SemaphoreType.DMA,
            pltpu.SemaphoreType.DMA,
            pltpu.SemaphoreType.DMA,
            pltpu.SemaphoreType.DMA,
            pltpu.SemaphoreType.DMA,
            pltpu.SemaphoreType.DMA,
            pltpu.SemaphoreType.DMA,
        ],
    )
    def k(dst_hbm, rec_hbm, xw2_hbm, xw1b_hbm, w3_hbm, out_hbm,
          acc, s0, s1, eidbuf, recb0, recb1, recb2, recb3,
          srcb0, srcb1, srcb2, srcb3, rowb0, rowb1, rowb2, rowb3,
          w3buf, xbuf,
          sem0, sem1, semr0, semr1, semw0, semw1,
          semg0, semg1, semg2, semg3, semx0, semx1, semx2, semx3):
        wid = lax.axis_index("s") * NC + lax.axis_index("c")
        pltpu.sync_copy(w3_hbm, w3buf)
        w3v = [[w3buf[pl.ds(r * 128 + v * 16, 16)] for v in range(8)]
               for r in range(4)]
        iota = lax.iota(I32, 16)
        col4 = jnp.full((16,), 4, I32)
        neg = jnp.full((16,), -3e38, F32)

        @pl.loop(0, nrounds)
        def _(rnd):
            lo = (rnd * NW + wid) * rng

            negb = plsc.pack(neg, neg, format=plsc.PackFormat.INTERLEAVED)

            @pl.loop(0, rng * 128 // 32)
            def _(i):
                acc[pl.ds(i * 32, 32)] = negb

            def extract_src(recb, srcb):
                for v in range(g // 16):
                    sf = plsc.load_gather(recb, [iota + v * 16, col4])
                    srcb[pl.ds(v * 16, 16)] = plsc.bitcast(sf, I32)

            def edge_loop(recb, rowsb):
                @pl.loop(0, g)
                def _(j):
                    prow = recb[j, pl.ds(0, 16)]
                    pint = plsc.bitcast(prow, I32)
                    dl = pint[5] - lo
                    p0 = prow[0]
                    p1 = prow[1]
                    p2 = prow[2]
                    p3 = prow[3]
                    ab = dl * 128
                    for v in range(4):
                        r0, r1 = plsc.unpack(
                            rowsb[j, pl.ds(v * 32, 32)],
                            format=plsc.PackFormat.INTERLEAVED)
                        u0 = (r0
                              + p0 * w3v[0][2 * v] + p1 * w3v[1][2 * v]
                              + p2 * w3v[2][2 * v] + p3 * w3v[3][2 * v])
                        u1 = (r1
                              + p0 * w3v[0][2 * v + 1]
                              + p1 * w3v[1][2 * v + 1]
                              + p2 * w3v[2][2 * v + 1]
                              + p3 * w3v[3][2 * v + 1])
                        sl = pl.ds(ab + v * 32, 32)
                        a0, a1 = plsc.unpack(
                            acc[sl], format=plsc.PackFormat.INTERLEAVED)
                        acc[sl] = plsc.pack(
                            jnp.maximum(a0, u0), jnp.maximum(a1, u1),
                            format=plsc.PackFormat.INTERLEAVED)

            recbs = [recb0, recb1, recb2, recb3]
            srcbs = [srcb0, srcb1, srcb2, srcb3]
            rowbs = [rowb0, rowb1, rowb2, rowb3]
            semgs = [semg0, semg1, semg2, semg3]
            semxs = [semx0, semx1, semx2, semx3]

            def process_group(i, carry):
                pltpu.sync_copy(
                    rec_hbm.at[eidbuf.at[pl.ds(i * g, g)]], recb0)
                extract_src(recb0, srcb0)
                pltpu.sync_copy(xw2_hbm.at[srcb0], rowb0)
                edge_loop(recb0, rowb0)
                return carry

            def process_quad(i, carry):
                base = i * 4 * g
                gs = [rec_hbm.at[eidbuf.at[pl.ds(base + kk * g, g)]]
                      for kk in range(4)]
                for kk in range(4):
                    pltpu.make_async_copy(gs[kk], recbs[kk],
                                          semgs[kk]).start()
                for kk in range(4):
                    pltpu.make_async_copy(gs[kk], recbs[kk],
                                          semgs[kk]).wait()
                    extract_src(recbs[kk], srcbs[kk])
                    pltpu.make_async_copy(
                        xw2_hbm.at[srcbs[kk]], rowbs[kk], semxs[kk]).start()
                for kk in range(4):
                    pltpu.make_async_copy(
                        xw2_hbm.at[srcbs[kk]], rowbs[kk], semxs[kk]).wait()
                    edge_loop(recbs[kk], rowbs[kk])
                return carry

            pltpu.make_async_copy(
                dst_hbm.at[pl.ds(0, ch)], s0, sem0).start()

            def chunk(c, n, sb, sem, sbn, semn):
                @pl.when(c + 1 < nchunk)
                def _():
                    pltpu.make_async_copy(
                        dst_hbm.at[pl.ds((c + 1) * ch, ch)], sbn,
                        semn).start()
                pltpu.make_async_copy(
                    dst_hbm.at[pl.ds(c * ch, ch)], sb, sem).wait()

                def scan_step(v, nn):
                    d = sb[pl.ds(v * 16, 16)]
                    m = plsc.bitcast(d - lo, jnp.uint32) < jnp.uint32(rng)
                    eidv = iota + (c * ch + v * 16)
                    plsc.store_compressed(eidbuf.at[pl.ds(nn, 16)], eidv,
                                          mask=m)
                    cnt = plsc.all_reduce_population_count(m)
                    return nn + cnt[0]

                n2 = lax.fori_loop(0, ch // 16, scan_step, n)
                nquads = n2 // (4 * g)
                lax.fori_loop(0, nquads, process_quad, 0)

                @pl.when(nquads > 0)
                def _():
                    for t in range(4 * g // 16):
                        eidbuf[pl.ds(t * 16, 16)] = (
                            eidbuf[pl.ds(nquads * 4 * g + t * 16, 16)])
                return n2 - nquads * 4 * g

            def pair(p, n):
                n = chunk(2 * p, n, s0, sem0, s1, sem1)
                n = chunk(2 * p + 1, n, s1, sem1, s0, sem0)
                return n

            n = lax.fori_loop(0, nchunk // 2, pair, 0)

            padv = jnp.full((16,), eidbuf[pl.ds(0, 16)][0], I32)
            for t in range(g // 16):
                eidbuf[pl.ds(n + t * 16, 16)] = padv
            lax.fori_loop(0, (n + g - 1) // g, process_group, 0)

            nfl = rng // 8
            rb = [xbuf.at[pl.ds(0, 8)], xbuf.at[pl.ds(8, 8)]]
            wb = [xbuf.at[pl.ds(16, 8)], xbuf.at[pl.ds(24, 8)]]
            semr = [semr0, semr1]
            semw = [semw0, semw1]
            for h in range(2):
                pltpu.make_async_copy(
                    xw1b_hbm.at[pl.ds(lo + h * 8, 8)], rb[h],
                    semr[h]).start()

            def flush_chunk(c, h):
                row0 = lo + c * 8
                pltpu.make_async_copy(
                    xw1b_hbm.at[pl.ds(row0, 8)], rb[h], semr[h]).wait()

                @pl.when(c >= 2)
                def _():
                    pltpu.make_async_copy(
                        wb[h], out_hbm.at[pl.ds(row0 - 16, 8)],
                        semw[h]).wait()

                @pl.loop(0, 8)
                def _(rr):
                    for v in range(4):
                        a0, a1 = plsc.unpack(
                            acc[pl.ds((c * 8 + rr) * 128 + v * 32, 32)],
                            format=plsc.PackFormat.INTERLEAVED)
                        sl0 = pl.ds(v * 32, 16)
                        sl1 = pl.ds(v * 32 + 16, 16)
                        wb[h][rr, sl0] = jnp.maximum(rb[h][rr, sl0] + a0,
                                                     0.0)
                        wb[h][rr, sl1] = jnp.maximum(rb[h][rr, sl1] + a1,
                                                     0.0)

                pltpu.make_async_copy(
                    wb[h], out_hbm.at[pl.ds(row0, 8)], semw[h]).start()

                @pl.when(c + 2 < nfl)
                def _():
                    pltpu.make_async_copy(
                        xw1b_hbm.at[pl.ds(row0 + 16, 8)], rb[h],
                        semr[h]).start()

            @pl.loop(0, nfl // 2)
            def _(t):
                flush_chunk(2 * t, 0)
                flush_chunk(2 * t + 1, 1)

            for h in range(2):
                pltpu.make_async_copy(
                    wb[h], out_hbm.at[pl.ds(lo + rng - 16 + h * 8, 8)],
                    semw[h]).wait()

    return k(dst, rec, xw2, xw1b, w3flat)


def _out_gather(idxp, outfull):
    nspad = idxp.shape[0]
    nsw = nspad // NW
    g = 400
    niter = nsw // g

    @functools.partial(
        pl.kernel,
        out_type=jax.ShapeDtypeStruct((nspad, 128), F32),
        mesh=_make_mesh(),
        compiler_params=_sc_params(),
        scratch_types=[
            pltpu.VMEM((g,), I32),
            pltpu.VMEM((g, 128), F32),
        ],
    )
    def k(idx_hbm, full_hbm, o_hbm, ibuf, rbuf):
        wid = lax.axis_index("s") * NC + lax.axis_index("c")

        @pl.loop(0, niter)
        def _(it):
            off = wid * nsw + it * g
            pltpu.sync_copy(idx_hbm.at[pl.ds(off, g)], ibuf)
            pltpu.sync_copy(full_hbm.at[ibuf], rbuf)
            pltpu.sync_copy(rbuf, o_hbm.at[pl.ds(off, g)])

    return k(idxp, outfull)


def kernel(x, pos, batch, norm, edge_index, idx, W, b):
    n, d = x.shape
    e = edge_index.shape[1]
    ns = idx.shape[0]
    assert d == 128 and n == 50000 and e == 400000 and ns == 12500

    npad = 50176
    epad = 409600
    nspad = 12800

    src = edge_index[0]
    dst = edge_index[1]

    p32 = jnp.array([32 * (j // 32) + (j % 32) // 2 + 16 * (j % 2)
                     for j in range(d)], dtype=I32)
    wc = jnp.concatenate([W[:d], W[d:2 * d][:, p32]], axis=1)
    w3flat = W[2 * d:2 * d + 4].reshape(-1)

    batchf = lax.bitcast_convert_type(batch, F32).reshape(n, 1)
    pn = jnp.concatenate(
        [pos, norm, batchf, jnp.zeros((n, 9), F32)], axis=1)

    x_pad = jnp.concatenate([x, jnp.zeros((npad - n, d), F32)], axis=0)
    srcp = jnp.concatenate([src, jnp.zeros((epad - e,), I32)])
    dstp = jnp.concatenate([dst, jnp.zeros((epad - e,), I32)])
    idxp = jnp.concatenate([idx, jnp.zeros((nspad - ns,), I32)])

    xw1b, xw2 = _matmul(x_pad, wc, b)
    ps_ext, pd_ext, pni = _gather_rows(pn, srcp, dstp, idxp)
    rec = _ppf(ps_ext.T, pd_ext.T).T
    outfull = _segmax(dst, rec, xw2, xw1b, w3flat)
    outg = _out_gather(idxp, outfull)

    x_out = outg[:ns]
    pos_out = pni[:ns, 0:3]
    batch_out = lax.bitcast_convert_type(pni[:ns, 6], I32)
    return (x_out, pos_out, batch_out, idx)
